# baseline (speedup 1.0000x reference)
import functools

import jax
import jax.numpy as jnp
from jax.experimental import pallas as pl
from jax.experimental.pallas import tpu as pltpu

DEXT = 144
_BLK = 2000


def _leaky(x):
    return jnp.where(x >= 0, x, 0.2 * x)


def _mm(a, b):
    return jax.lax.dot_general(
        a, b, (((1,), (0,)), ((), ())),
        precision=jax.lax.Precision.HIGHEST,
        preferred_element_type=jnp.float32,
    )


def _ext_block(sx, alpha_src):
    B = sx.shape[0]
    lane = jax.lax.broadcasted_iota(jnp.int32, (B, DEXT - 128), 1)
    extra = jnp.where(lane == 0, 1.0, jnp.where(lane == 1, alpha_src, 0.0))
    return jnp.concatenate([sx, extra.astype(jnp.float32)], axis=1)


def _proj_pair(xs, xd, Wsrc, Wdst, a):
    sx = _mm(xs, Wsrc)
    tx = _mm(xd, Wdst)
    asrc = _mm(sx, a[:128])
    adst = _mm(tx, a[128:])
    return _ext_block(sx, asrc), tx, jnp.broadcast_to(adst, (adst.shape[0], 8))


def _layer0_body(xu_ref, xi_ref, wsu_ref, wdu_ref, au_ref, wsi_ref, wdi_ref,
                 ai_ref, sxe_ui_ref, tx_ui_ref, at_ui_ref, sxe_iu_ref,
                 tx_iu_ref, at_iu_ref):
    xu = xu_ref[...]
    xi = xi_ref[...]
    sxe_ui_ref[...], tx_ui_ref[...], at_ui_ref[...] = _proj_pair(
        xu, xi, wsu_ref[...], wdu_ref[...], au_ref[...])
    sxe_iu_ref[...], tx_iu_ref[...], at_iu_ref[...] = _proj_pair(
        xi, xu, wsi_ref[...], wdi_ref[...], ai_ref[...])


def _epilogue(tx, agg):
    return jax.nn.relu(tx + agg[:, :128] / (agg[:, 128:129] + 1e-06))


def _layer1_body(txp_ui_ref, agg_ui_ref, txp_iu_ref, agg_iu_ref, wsu_ref,
                 wdu_ref, au_ref, wsi_ref, wdi_ref, ai_ref, sxe_ui_ref,
                 tx_ui_ref, at_ui_ref, sxe_iu_ref, tx_iu_ref, at_iu_ref):
    xi = _epilogue(txp_ui_ref[...], agg_ui_ref[...])
    xu = _epilogue(txp_iu_ref[...], agg_iu_ref[...])
    sxe_ui_ref[...], tx_ui_ref[...], at_ui_ref[...] = _proj_pair(
        xu, xi, wsu_ref[...], wdu_ref[...], au_ref[...])
    sxe_iu_ref[...], tx_iu_ref[...], at_iu_ref[...] = _proj_pair(
        xi, xu, wsi_ref[...], wdi_ref[...], ai_ref[...])


def _final_body(txp_ui_ref, agg_ui_ref, txp_iu_ref, agg_iu_ref, wu_ref,
                bu_ref, wi_ref, bi_ref, xu_ref, xi_ref, ou_ref, oi_ref):
    xi = _epilogue(txp_ui_ref[...], agg_ui_ref[...])
    xu = _epilogue(txp_iu_ref[...], agg_iu_ref[...])
    xu_ref[...] = xu
    xi_ref[...] = xi
    ou_ref[...] = _mm(xu, wu_ref[...]) + bu_ref[...]
    oi_ref[...] = _mm(xi, wi_ref[...]) + bi_ref[...]


def _row_spec(width):
    return pl.BlockSpec((_BLK, width), lambda i: (i, 0))


def _full_spec(shape):
    return pl.BlockSpec(shape, lambda i: tuple(0 for _ in shape))


def _run_layer_matmuls(body, row_inputs, weight_inputs, n):
    grid = n // _BLK
    row_widths = [x.shape[1] for x in row_inputs]
    in_specs = [_row_spec(w) for w in row_widths]
    in_specs += [_full_spec(w.shape) for w in weight_inputs]
    out_shape = [
        jax.ShapeDtypeStruct((n, DEXT), jnp.float32),
        jax.ShapeDtypeStruct((n, 128), jnp.float32),
        jax.ShapeDtypeStruct((n, 8), jnp.float32),
        jax.ShapeDtypeStruct((n, DEXT), jnp.float32),
        jax.ShapeDtypeStruct((n, 128), jnp.float32),
        jax.ShapeDtypeStruct((n, 8), jnp.float32),
    ]
    out_specs = [_row_spec(s.shape[1]) for s in out_shape]
    return pl.pallas_call(
        body, grid=(grid,), in_specs=in_specs, out_specs=out_specs,
        out_shape=out_shape,
    )(*row_inputs, *weight_inputs)


def _run_final(row_inputs, weight_inputs, n):
    in_specs = [_row_spec(x.shape[1]) for x in row_inputs]
    in_specs += [_full_spec(w.shape) for w in weight_inputs]
    out_shape = [
        jax.ShapeDtypeStruct((n, 128), jnp.float32),
        jax.ShapeDtypeStruct((n, 128), jnp.float32),
        jax.ShapeDtypeStruct((n, 64), jnp.float32),
        jax.ShapeDtypeStruct((n, 64), jnp.float32),
    ]
    out_specs = [_row_spec(s.shape[1]) for s in out_shape]
    return pl.pallas_call(
        _final_body, grid=(n // _BLK,), in_specs=in_specs,
        out_specs=out_specs, out_shape=out_shape,
    )(*row_inputs, *weight_inputs)


def _conv_edges(sxe, alpha_t, si, ti, n):
    s2 = si[si]
    att = jnp.exp(_leaky(sxe[s2, 129] + alpha_t[ti]))
    msg = sxe[s2] * att[:, None]
    return jax.ops.segment_sum(msg, ti, num_segments=n)


def kernel(x_user, x_item, edge_index_ui, edge_index_iu, params):
    n = x_user.shape[0]
    p = params
    si_ui, ti_ui = edge_index_ui[0], edge_index_ui[1]
    si_iu, ti_iu = edge_index_iu[0], edge_index_iu[1]

    w0 = [p["l0_ui_Wsrc"], p["l0_ui_Wdst"], p["l0_ui_a"],
          p["l0_iu_Wsrc"], p["l0_iu_Wdst"], p["l0_iu_a"]]
    sxe_ui, tx_ui, at_ui, sxe_iu, tx_iu, at_iu = _run_layer_matmuls(
        _layer0_body, [x_user, x_item], w0, n)

    agg_ui = _conv_edges(sxe_ui, at_ui[:, 0], si_ui, ti_ui, n)
    agg_iu = _conv_edges(sxe_iu, at_iu[:, 0], si_iu, ti_iu, n)

    w1 = [p["l1_ui_Wsrc"], p["l1_ui_Wdst"], p["l1_ui_a"],
          p["l1_iu_Wsrc"], p["l1_iu_Wdst"], p["l1_iu_a"]]
    sxe_ui, tx_ui, at_ui, sxe_iu, tx_iu, at_iu = _run_layer_matmuls(
        _layer1_body, [tx_ui, agg_ui, tx_iu, agg_iu], w1, n)

    agg_ui = _conv_edges(sxe_ui, at_ui[:, 0], si_ui, ti_ui, n)
    agg_iu = _conv_edges(sxe_iu, at_iu[:, 0], si_iu, ti_iu, n)

    wf = [p["lin_user_W"], p["lin_user_b"], p["lin_item_W"], p["lin_item_b"]]
    xu2, xi2, out_u, out_i = _run_final(
        [tx_ui, agg_ui, tx_iu, agg_iu], wf, n)
    return (xu2, xi2, out_u, out_i)


# speedup vs baseline: 5.9149x; 5.9149x over previous
import functools

import jax
import jax.numpy as jnp
from jax import lax
from jax.experimental import pallas as pl
from jax.experimental.pallas import tpu as pltpu
from jax.experimental.pallas import tpu_sc as plsc

_BLK = 2000


def _mm(a, b):
    return jax.lax.dot_general(
        a, b, (((1,), (0,)), ((), ())),
        precision=jax.lax.Precision.HIGHEST,
        preferred_element_type=jnp.float32,
    )


def _proj_pair(xs, xd, Wsrc, Wdst, a):
    sx = _mm(xs, Wsrc)
    tx = _mm(xd, Wdst)
    asrc = _mm(sx, a[:128])
    adst = _mm(tx, a[128:])
    B = sx.shape[0]
    return sx, tx, jnp.broadcast_to(asrc, (B, 8)), jnp.broadcast_to(adst, (B, 8))


def _layer0_body(xu_ref, xi_ref, wsu_ref, wdu_ref, au_ref, wsi_ref, wdi_ref,
                 ai_ref, sx_ui_ref, tx_ui_ref, as_ui_ref, at_ui_ref,
                 sx_iu_ref, tx_iu_ref, as_iu_ref, at_iu_ref):
    xu = xu_ref[...]
    xi = xi_ref[...]
    sx_ui_ref[...], tx_ui_ref[...], as_ui_ref[...], at_ui_ref[...] = (
        _proj_pair(xu, xi, wsu_ref[...], wdu_ref[...], au_ref[...]))
    sx_iu_ref[...], tx_iu_ref[...], as_iu_ref[...], at_iu_ref[...] = (
        _proj_pair(xi, xu, wsi_ref[...], wdi_ref[...], ai_ref[...]))


def _layer1_body(txp_ui_ref, agg_ui_ref, txp_iu_ref, agg_iu_ref, wsu_ref,
                 wdu_ref, au_ref, wsi_ref, wdi_ref, ai_ref, sx_ui_ref,
                 tx_ui_ref, as_ui_ref, at_ui_ref, sx_iu_ref, tx_iu_ref,
                 as_iu_ref, at_iu_ref):
    xi = jax.nn.relu(txp_ui_ref[...] + agg_ui_ref[...])
    xu = jax.nn.relu(txp_iu_ref[...] + agg_iu_ref[...])
    sx_ui_ref[...], tx_ui_ref[...], as_ui_ref[...], at_ui_ref[...] = (
        _proj_pair(xu, xi, wsu_ref[...], wdu_ref[...], au_ref[...]))
    sx_iu_ref[...], tx_iu_ref[...], as_iu_ref[...], at_iu_ref[...] = (
        _proj_pair(xi, xu, wsi_ref[...], wdi_ref[...], ai_ref[...]))


def _final_body(txp_ui_ref, agg_ui_ref, txp_iu_ref, agg_iu_ref, wu_ref,
                bu_ref, wi_ref, bi_ref, xu_ref, xi_ref, ou_ref, oi_ref):
    xi = jax.nn.relu(txp_ui_ref[...] + agg_ui_ref[...])
    xu = jax.nn.relu(txp_iu_ref[...] + agg_iu_ref[...])
    xu_ref[...] = xu
    xi_ref[...] = xi
    ou_ref[...] = _mm(xu, wu_ref[...]) + bu_ref[...]
    oi_ref[...] = _mm(xi, wi_ref[...]) + bi_ref[...]


def _row_spec(width):
    return pl.BlockSpec((_BLK, width), lambda i: (i, 0))


def _full_spec(shape):
    return pl.BlockSpec(shape, lambda i: tuple(0 for _ in shape))


def _run_layer_matmuls(body, row_inputs, weight_inputs, n):
    in_specs = [_row_spec(x.shape[1]) for x in row_inputs]
    in_specs += [_full_spec(w.shape) for w in weight_inputs]
    out_shape = [
        jax.ShapeDtypeStruct((n, 128), jnp.float32),
        jax.ShapeDtypeStruct((n, 128), jnp.float32),
        jax.ShapeDtypeStruct((n, 8), jnp.float32),
        jax.ShapeDtypeStruct((n, 8), jnp.float32),
    ] * 2
    out_specs = [_row_spec(s.shape[1]) for s in out_shape]
    return pl.pallas_call(
        body, grid=(n // _BLK,), in_specs=in_specs, out_specs=out_specs,
        out_shape=out_shape,
    )(*row_inputs, *weight_inputs)


def _run_final(row_inputs, weight_inputs, n):
    in_specs = [_row_spec(x.shape[1]) for x in row_inputs]
    in_specs += [_full_spec(w.shape) for w in weight_inputs]
    out_shape = [
        jax.ShapeDtypeStruct((n, 128), jnp.float32),
        jax.ShapeDtypeStruct((n, 128), jnp.float32),
        jax.ShapeDtypeStruct((n, 64), jnp.float32),
        jax.ShapeDtypeStruct((n, 64), jnp.float32),
    ]
    out_specs = [_row_spec(s.shape[1]) for s in out_shape]
    return pl.pallas_call(
        _final_body, grid=(n // _BLK,), in_specs=in_specs,
        out_specs=out_specs, out_shape=out_shape,
    )(*row_inputs, *weight_inputs)


_C = 128
_NSUB = 16
_WB = 80

_SC_PARAMS = pltpu.CompilerParams(
    needs_layout_passes=False, use_tc_tiling_on_sc=False)
_MESH = plsc.VectorSubcoreMesh(core_axis_name="c", subcore_axis_name="s")


def _for_each_wb_chunk(s, nwb, fn):
    def body(b, _):
        cid = s + _NSUB * b

        @pl.when(cid < nwb)
        def _():
            fn(cid * _WB)
        return 0

    lax.fori_loop(0, (nwb + _NSUB - 1) // _NSUB, body, 0)


def _sc_att_layer(zeros8, as_ui, at_ui, si_ui, ti_ui, as_iu, at_iu, si_iu,
                  ti_iu):
    n = as_ui.shape[0]
    e_cnt = si_ui.shape[0]
    nchunks = e_cnt // _C
    assert e_cnt % _C == 0 and n % _WB == 0
    nwb = n // _WB
    kmax = (nchunks + _NSUB - 1) // _NSUB

    @functools.partial(
        pl.kernel, mesh=_MESH, compiler_params=_SC_PARAMS,
        out_type=[
            jax.ShapeDtypeStruct((e_cnt,), jnp.int32),
            jax.ShapeDtypeStruct((e_cnt,), jnp.float32),
            jax.ShapeDtypeStruct((n,), jnp.float32),
            jax.ShapeDtypeStruct((e_cnt,), jnp.int32),
            jax.ShapeDtypeStruct((e_cnt,), jnp.float32),
            jax.ShapeDtypeStruct((n,), jnp.float32),
        ],
        scratch_types=[
            pltpu.VMEM((n,), jnp.int32),
            pltpu.VMEM((n,), jnp.float32),
            pltpu.VMEM((n,), jnp.float32),
            pltpu.VMEM((_C,), jnp.int32),
            pltpu.VMEM((_C,), jnp.int32),
            pltpu.VMEM((_C,), jnp.int32),
            pltpu.VMEM((_C,), jnp.float32),
            pltpu.VMEM((_C, 8), jnp.float32),
            pltpu.VMEM((_WB, 8), jnp.float32),
            pltpu.VMEM((_WB,), jnp.float32),
            pltpu.VMEM_SHARED((n, 8), jnp.float32),
        ])
    def att_kernel(z8_h, as_ui_h, at_ui_h, si_ui_h, ti_ui_h, as_iu_h,
                   at_iu_h, si_iu_h, ti_iu_h, s2_ui_h, att_ui_h, dv_ui_h,
                   s2_iu_h, att_iu_h, dv_iu_h, si_tab, as_tab, at_tab, si_c,
                   ti_c, s2_c, att_c, att8, bounce, dvc, dv_sh):
        c = lax.axis_index("c")
        s = lax.axis_index("s")
        iota16 = lax.broadcasted_iota(jnp.int32, (16,), 0)

        pltpu.sync_copy(z8_h.at[pl.ds(0, _WB)], bounce)
        pltpu.sync_copy(z8_h, att8)
        _for_each_wb_chunk(
            s, nwb, lambda r0: pltpu.sync_copy(
                bounce, dv_sh.at[pl.ds(r0, _WB)]))
        plsc.subcore_barrier()

        def process(as_h, at_h, si_h, ti_h, s2_h, att_h):
            pltpu.sync_copy(si_h.at[pl.ds(0, n)], si_tab)
            pltpu.sync_copy(as_h, as_tab)
            pltpu.sync_copy(at_h, at_tab)

            def chunk_body(k, _):
                cid = s + _NSUB * k

                @pl.when(cid < nchunks)
                def _():
                    off = cid * _C
                    pltpu.sync_copy(si_h.at[pl.ds(off, _C)], si_c)
                    pltpu.sync_copy(ti_h.at[pl.ds(off, _C)], ti_c)
                    for i in range(_C // 16):
                        sl = pl.ds(16 * i, 16)
                        e16 = iota16 + 16 * i
                        s2_16 = plsc.load_gather(si_tab, [si_c[sl]])
                        s2_c[sl] = s2_16
                        x = (plsc.load_gather(as_tab, [s2_16])
                             + plsc.load_gather(at_tab, [ti_c[sl]]))
                        att16 = jnp.exp(jnp.where(x >= 0, x, 0.2 * x))
                        att_c[sl] = att16
                        plsc.store_scatter(
                            att8, [e16, jnp.zeros((16,), jnp.int32)], att16)
                    pltpu.sync_copy(s2_c, s2_h.at[pl.ds(off, _C)])
                    pltpu.sync_copy(att_c, att_h.at[pl.ds(off, _C)])
                    pltpu.sync_copy(att8, dv_sh.at[ti_c], add=True)
                return 0

            lax.fori_loop(0, kmax, chunk_body, 0)

        @pl.when(c == 0)
        def _():
            process(as_ui_h, at_ui_h, si_ui_h, ti_ui_h, s2_ui_h, att_ui_h)

        @pl.when(c == 1)
        def _():
            process(as_iu_h, at_iu_h, si_iu_h, ti_iu_h, s2_iu_h, att_iu_h)

        plsc.subcore_barrier()

        def compact_out(r0):
            pltpu.sync_copy(dv_sh.at[pl.ds(r0, _WB)], bounce)
            for j in range(_WB // 16):
                e16 = iota16 + 16 * j
                dvc[pl.ds(16 * j, 16)] = plsc.load_gather(
                    bounce, [e16, jnp.zeros((16,), jnp.int32)])

            @pl.when(c == 0)
            def _():
                pltpu.sync_copy(dvc, dv_ui_h.at[pl.ds(r0, _WB)])

            @pl.when(c == 1)
            def _():
                pltpu.sync_copy(dvc, dv_iu_h.at[pl.ds(r0, _WB)])

        _for_each_wb_chunk(s, nwb, compact_out)

    return att_kernel(zeros8, as_ui, at_ui, si_ui, ti_ui,
                      as_iu, at_iu, si_iu, ti_iu)


def _sc_agg_layer(sx_ui, att_ui, dv_ui, s2_ui, ti_ui,
                  sx_iu, att_iu, dv_iu, s2_iu, ti_iu):
    n = sx_ui.shape[0]
    e_cnt = s2_ui.shape[0]
    nchunks = e_cnt // _C
    nwb = n // _WB
    kmax = (nchunks + _NSUB - 1) // _NSUB

    @functools.partial(
        pl.kernel, mesh=_MESH, compiler_params=_SC_PARAMS,
        out_type=[jax.ShapeDtypeStruct((n, 128), jnp.float32)] * 2,
        scratch_types=[
            pltpu.VMEM((n,), jnp.float32),
            pltpu.VMEM((_C,), jnp.int32),
            pltpu.VMEM((_C,), jnp.int32),
            pltpu.VMEM((_C,), jnp.float32),
            pltpu.VMEM((_C, 128), jnp.float32),
            pltpu.VMEM((_WB, 128), jnp.float32),
            pltpu.VMEM_SHARED((n, 128), jnp.float32),
            pltpu.SemaphoreType.DMA,
        ])
    def agg_kernel(sx_ui_h, att_ui_h, dv_ui_h, s2_ui_h, ti_ui_h, sx_iu_h,
                   att_iu_h, dv_iu_h, s2_iu_h, ti_iu_h, agg_ui_h, agg_iu_h,
                   inv_tab, ti_c, s2_c, w_c, rows, bounce, agg_sh, sem):
        c = lax.axis_index("c")
        s = lax.axis_index("s")
        iota16 = lax.broadcasted_iota(jnp.int32, (16,), 0)

        def zrow(r, _):
            for j in range(8):
                bounce[r, pl.ds(16 * j, 16)] = jnp.zeros((16,), jnp.float32)
            return 0

        lax.fori_loop(0, _WB, zrow, 0)
        _for_each_wb_chunk(
            s, nwb, lambda r0: pltpu.sync_copy(
                bounce, agg_sh.at[pl.ds(r0, _WB)]))
        plsc.subcore_barrier()

        def process(sx_h, att_h, dv_h, s2_h, ti_h):
            pltpu.sync_copy(dv_h, inv_tab)

            def inv_body(j, _):
                sl = pl.ds(16 * j, 16)
                inv_tab[sl] = 1.0 / (inv_tab[sl] + 1e-06)
                return 0

            lax.fori_loop(0, n // 16, inv_body, 0)

            def chunk_body(k, _):
                cid = s + _NSUB * k

                @pl.when(cid < nchunks)
                def _():
                    off = cid * _C
                    pltpu.sync_copy(ti_h.at[pl.ds(off, _C)], ti_c)
                    pltpu.sync_copy(s2_h.at[pl.ds(off, _C)], s2_c)
                    pltpu.sync_copy(att_h.at[pl.ds(off, _C)], w_c)
                    for i in range(_C // 16):
                        sl = pl.ds(16 * i, 16)
                        w_c[sl] = w_c[sl] * plsc.load_gather(
                            inv_tab, [ti_c[sl]])
                    pltpu.async_copy(sx_h.at[s2_c], rows, sem).wait()

                    def scale_body(e, _):
                        bvec = plsc.load_gather(
                            w_c, [jnp.full((16,), e, jnp.int32)])
                        for r in range(8):
                            sl = pl.ds(16 * r, 16)
                            rows[e, sl] = rows[e, sl] * bvec
                        return 0

                    lax.fori_loop(0, _C, scale_body, 0)
                    pltpu.sync_copy(rows, agg_sh.at[ti_c], add=True)
                return 0

            lax.fori_loop(0, kmax, chunk_body, 0)

        @pl.when(c == 0)
        def _():
            process(sx_ui_h, att_ui_h, dv_ui_h, s2_ui_h, ti_ui_h)

        @pl.when(c == 1)
        def _():
            process(sx_iu_h, att_iu_h, dv_iu_h, s2_iu_h, ti_iu_h)

        plsc.subcore_barrier()

        def wb(r0):
            pltpu.sync_copy(agg_sh.at[pl.ds(r0, _WB)], bounce)

            @pl.when(c == 0)
            def _():
                pltpu.sync_copy(bounce, agg_ui_h.at[pl.ds(r0, _WB)])

            @pl.when(c == 1)
            def _():
                pltpu.sync_copy(bounce, agg_iu_h.at[pl.ds(r0, _WB)])

        _for_each_wb_chunk(s, nwb, wb)

    return agg_kernel(sx_ui, att_ui, dv_ui, s2_ui, ti_ui,
                      sx_iu, att_iu, dv_iu, s2_iu, ti_iu)


def _sc_conv_layer(zeros8, sx_ui, as_ui, at_ui, si_ui, ti_ui,
                   sx_iu, as_iu, at_iu, si_iu, ti_iu):
    s2_ui, att_ui, dv_ui, s2_iu, att_iu, dv_iu = _sc_att_layer(
        zeros8, as_ui, at_ui, si_ui, ti_ui, as_iu, at_iu, si_iu, ti_iu)
    return _sc_agg_layer(sx_ui, att_ui, dv_ui, s2_ui, ti_ui,
                         sx_iu, att_iu, dv_iu, s2_iu, ti_iu)


def kernel(x_user, x_item, edge_index_ui, edge_index_iu, params):
    n = x_user.shape[0]
    p = params
    si_ui, ti_ui = edge_index_ui[0], edge_index_ui[1]
    si_iu, ti_iu = edge_index_iu[0], edge_index_iu[1]

    w0 = [p["l0_ui_Wsrc"], p["l0_ui_Wdst"], p["l0_ui_a"],
          p["l0_iu_Wsrc"], p["l0_iu_Wdst"], p["l0_iu_a"]]
    (sx_ui, tx_ui, as_ui, at_ui, sx_iu, tx_iu, as_iu, at_iu) = (
        _run_layer_matmuls(_layer0_body, [x_user, x_item], w0, n))

    zeros8 = jnp.zeros((_C, 8), jnp.float32)
    agg_ui, agg_iu = _sc_conv_layer(
        zeros8, sx_ui, as_ui[:, 0], at_ui[:, 0], si_ui, ti_ui,
        sx_iu, as_iu[:, 0], at_iu[:, 0], si_iu, ti_iu)

    w1 = [p["l1_ui_Wsrc"], p["l1_ui_Wdst"], p["l1_ui_a"],
          p["l1_iu_Wsrc"], p["l1_iu_Wdst"], p["l1_iu_a"]]
    (sx_ui, tx_ui, as_ui, at_ui, sx_iu, tx_iu, as_iu, at_iu) = (
        _run_layer_matmuls(_layer1_body, [tx_ui, agg_ui, tx_iu, agg_iu],
                           w1, n))

    agg_ui, agg_iu = _sc_conv_layer(
        zeros8, sx_ui, as_ui[:, 0], at_ui[:, 0], si_ui, ti_ui,
        sx_iu, as_iu[:, 0], at_iu[:, 0], si_iu, ti_iu)

    wf = [p["lin_user_W"], p["lin_user_b"], p["lin_item_W"], p["lin_item_b"]]
    xu2, xi2, out_u, out_i = _run_final(
        [tx_ui, agg_ui, tx_iu, agg_iu], wf, n)
    return (xu2, xi2, out_u, out_i)


# speedup vs baseline: 8.4206x; 1.4236x over previous
import functools

import jax
import jax.numpy as jnp
from jax import lax
from jax.experimental import pallas as pl
from jax.experimental.pallas import tpu as pltpu
from jax.experimental.pallas import tpu_sc as plsc

_BLK = 2000


def _mm(a, b):
    return jax.lax.dot_general(
        a, b, (((1,), (0,)), ((), ())),
        precision=jax.lax.Precision.HIGHEST,
        preferred_element_type=jnp.float32,
    )


def _proj_pair(xs, xd, Wsrc, Wdst, a):
    sx = _mm(xs, Wsrc)
    tx = _mm(xd, Wdst)
    asrc = _mm(sx, a[:128])
    adst = _mm(tx, a[128:])
    B = sx.shape[0]
    return sx, tx, jnp.broadcast_to(asrc, (B, 8)), jnp.broadcast_to(adst, (B, 8))


def _layer0_body(xu_ref, xi_ref, wsu_ref, wdu_ref, au_ref, wsi_ref, wdi_ref,
                 ai_ref, sx_ui_ref, tx_ui_ref, as_ui_ref, at_ui_ref,
                 sx_iu_ref, tx_iu_ref, as_iu_ref, at_iu_ref):
    xu = xu_ref[...]
    xi = xi_ref[...]
    sx_ui_ref[...], tx_ui_ref[...], as_ui_ref[...], at_ui_ref[...] = (
        _proj_pair(xu, xi, wsu_ref[...], wdu_ref[...], au_ref[...]))
    sx_iu_ref[...], tx_iu_ref[...], as_iu_ref[...], at_iu_ref[...] = (
        _proj_pair(xi, xu, wsi_ref[...], wdi_ref[...], ai_ref[...]))


def _layer1_body(txp_ui_ref, agg_ui_ref, txp_iu_ref, agg_iu_ref, wsu_ref,
                 wdu_ref, au_ref, wsi_ref, wdi_ref, ai_ref, sx_ui_ref,
                 tx_ui_ref, as_ui_ref, at_ui_ref, sx_iu_ref, tx_iu_ref,
                 as_iu_ref, at_iu_ref):
    xi = jax.nn.relu(txp_ui_ref[...] + agg_ui_ref[...])
    xu = jax.nn.relu(txp_iu_ref[...] + agg_iu_ref[...])
    sx_ui_ref[...], tx_ui_ref[...], as_ui_ref[...], at_ui_ref[...] = (
        _proj_pair(xu, xi, wsu_ref[...], wdu_ref[...], au_ref[...]))
    sx_iu_ref[...], tx_iu_ref[...], as_iu_ref[...], at_iu_ref[...] = (
        _proj_pair(xi, xu, wsi_ref[...], wdi_ref[...], ai_ref[...]))


def _final_body(txp_ui_ref, agg_ui_ref, txp_iu_ref, agg_iu_ref, wu_ref,
                bu_ref, wi_ref, bi_ref, xu_ref, xi_ref, ou_ref, oi_ref):
    xi = jax.nn.relu(txp_ui_ref[...] + agg_ui_ref[...])
    xu = jax.nn.relu(txp_iu_ref[...] + agg_iu_ref[...])
    xu_ref[...] = xu
    xi_ref[...] = xi
    ou_ref[...] = _mm(xu, wu_ref[...]) + bu_ref[...]
    oi_ref[...] = _mm(xi, wi_ref[...]) + bi_ref[...]


def _row_spec(width):
    return pl.BlockSpec((_BLK, width), lambda i: (i, 0))


def _full_spec(shape):
    return pl.BlockSpec(shape, lambda i: tuple(0 for _ in shape))


def _run_layer_matmuls(body, row_inputs, weight_inputs, n):
    in_specs = [_row_spec(x.shape[1]) for x in row_inputs]
    in_specs += [_full_spec(w.shape) for w in weight_inputs]
    out_shape = [
        jax.ShapeDtypeStruct((n, 128), jnp.float32),
        jax.ShapeDtypeStruct((n, 128), jnp.float32),
        jax.ShapeDtypeStruct((n, 8), jnp.float32),
        jax.ShapeDtypeStruct((n, 8), jnp.float32),
    ] * 2
    out_specs = [_row_spec(s.shape[1]) for s in out_shape]
    return pl.pallas_call(
        body, grid=(n // _BLK,), in_specs=in_specs, out_specs=out_specs,
        out_shape=out_shape,
    )(*row_inputs, *weight_inputs)


def _run_final(row_inputs, weight_inputs, n):
    in_specs = [_row_spec(x.shape[1]) for x in row_inputs]
    in_specs += [_full_spec(w.shape) for w in weight_inputs]
    out_shape = [
        jax.ShapeDtypeStruct((n, 128), jnp.float32),
        jax.ShapeDtypeStruct((n, 128), jnp.float32),
        jax.ShapeDtypeStruct((n, 64), jnp.float32),
        jax.ShapeDtypeStruct((n, 64), jnp.float32),
    ]
    out_specs = [_row_spec(s.shape[1]) for s in out_shape]
    return pl.pallas_call(
        _final_body, grid=(n // _BLK,), in_specs=in_specs,
        out_specs=out_specs, out_shape=out_shape,
    )(*row_inputs, *weight_inputs)


_C = 128
_NSUB = 16
_WB = 80

_SC_PARAMS = pltpu.CompilerParams(
    needs_layout_passes=False, use_tc_tiling_on_sc=False)
_MESH = plsc.VectorSubcoreMesh(core_axis_name="c", subcore_axis_name="s")


def _for_each_wb_chunk(s, nwb, fn):
    def body(b, _):
        cid = s + _NSUB * b

        @pl.when(cid < nwb)
        def _():
            fn(cid * _WB)
        return 0

    lax.fori_loop(0, (nwb + _NSUB - 1) // _NSUB, body, 0)


def _sc_att_layer(zeros8, as_ui, at_ui, si_ui, ti_ui, as_iu, at_iu, si_iu,
                  ti_iu):
    n = as_ui.shape[0]
    e_cnt = si_ui.shape[0]
    nchunks = e_cnt // _C
    assert e_cnt % _C == 0 and n % _WB == 0
    nwb = n // _WB
    kmax = (nchunks + _NSUB - 1) // _NSUB

    @functools.partial(
        pl.kernel, mesh=_MESH, compiler_params=_SC_PARAMS,
        out_type=[
            jax.ShapeDtypeStruct((e_cnt,), jnp.int32),
            jax.ShapeDtypeStruct((e_cnt,), jnp.float32),
            jax.ShapeDtypeStruct((n,), jnp.float32),
            jax.ShapeDtypeStruct((e_cnt,), jnp.int32),
            jax.ShapeDtypeStruct((e_cnt,), jnp.float32),
            jax.ShapeDtypeStruct((n,), jnp.float32),
        ],
        scratch_types=[
            pltpu.VMEM((n,), jnp.int32),
            pltpu.VMEM((n,), jnp.float32),
            pltpu.VMEM((n,), jnp.float32),
            pltpu.VMEM((_C,), jnp.int32),
            pltpu.VMEM((_C,), jnp.int32),
            pltpu.VMEM((_C,), jnp.int32),
            pltpu.VMEM((_C,), jnp.float32),
            pltpu.VMEM((_C, 8), jnp.float32),
            pltpu.VMEM((_WB, 8), jnp.float32),
            pltpu.VMEM((_WB,), jnp.float32),
            pltpu.VMEM_SHARED((n, 8), jnp.float32),
        ])
    def att_kernel(z8_h, as_ui_h, at_ui_h, si_ui_h, ti_ui_h, as_iu_h,
                   at_iu_h, si_iu_h, ti_iu_h, s2_ui_h, att_ui_h, dv_ui_h,
                   s2_iu_h, att_iu_h, dv_iu_h, si_tab, as_tab, at_tab, si_c,
                   ti_c, s2_c, att_c, att8, bounce, dvc, dv_sh):
        c = lax.axis_index("c")
        s = lax.axis_index("s")
        iota16 = lax.broadcasted_iota(jnp.int32, (16,), 0)

        pltpu.sync_copy(z8_h.at[pl.ds(0, _WB)], bounce)
        pltpu.sync_copy(z8_h, att8)
        _for_each_wb_chunk(
            s, nwb, lambda r0: pltpu.sync_copy(
                bounce, dv_sh.at[pl.ds(r0, _WB)]))
        plsc.subcore_barrier()

        def process(as_h, at_h, si_h, ti_h, s2_h, att_h):
            pltpu.sync_copy(si_h.at[pl.ds(0, n)], si_tab)
            pltpu.sync_copy(as_h, as_tab)
            pltpu.sync_copy(at_h, at_tab)

            def chunk_body(k, _):
                cid = s + _NSUB * k

                @pl.when(cid < nchunks)
                def _():
                    off = cid * _C
                    pltpu.sync_copy(si_h.at[pl.ds(off, _C)], si_c)
                    pltpu.sync_copy(ti_h.at[pl.ds(off, _C)], ti_c)
                    for i in range(_C // 16):
                        sl = pl.ds(16 * i, 16)
                        e16 = iota16 + 16 * i
                        s2_16 = plsc.load_gather(si_tab, [si_c[sl]])
                        s2_c[sl] = s2_16
                        x = (plsc.load_gather(as_tab, [s2_16])
                             + plsc.load_gather(at_tab, [ti_c[sl]]))
                        att16 = jnp.exp(jnp.where(x >= 0, x, 0.2 * x))
                        att_c[sl] = att16
                        plsc.store_scatter(
                            att8, [e16, jnp.zeros((16,), jnp.int32)], att16)
                    pltpu.sync_copy(s2_c, s2_h.at[pl.ds(off, _C)])
                    pltpu.sync_copy(att_c, att_h.at[pl.ds(off, _C)])
                    pltpu.sync_copy(att8, dv_sh.at[ti_c], add=True)
                return 0

            lax.fori_loop(0, kmax, chunk_body, 0)

        @pl.when(c == 0)
        def _():
            process(as_ui_h, at_ui_h, si_ui_h, ti_ui_h, s2_ui_h, att_ui_h)

        @pl.when(c == 1)
        def _():
            process(as_iu_h, at_iu_h, si_iu_h, ti_iu_h, s2_iu_h, att_iu_h)

        plsc.subcore_barrier()

        def compact_out(r0):
            pltpu.sync_copy(dv_sh.at[pl.ds(r0, _WB)], bounce)
            for j in range(_WB // 16):
                e16 = iota16 + 16 * j
                dvc[pl.ds(16 * j, 16)] = plsc.load_gather(
                    bounce, [e16, jnp.zeros((16,), jnp.int32)])

            @pl.when(c == 0)
            def _():
                pltpu.sync_copy(dvc, dv_ui_h.at[pl.ds(r0, _WB)])

            @pl.when(c == 1)
            def _():
                pltpu.sync_copy(dvc, dv_iu_h.at[pl.ds(r0, _WB)])

        _for_each_wb_chunk(s, nwb, compact_out)

    return att_kernel(zeros8, as_ui, at_ui, si_ui, ti_ui,
                      as_iu, at_iu, si_iu, ti_iu)


_CB = 64
_SUB = 20
_SS = _SUB * _CB


def _sc_agg_layer(sx_ui, att_ui, dv_ui, s2_ui, ti_ui,
                  sx_iu, att_iu, dv_iu, s2_iu, ti_iu):
    n = sx_ui.shape[0]
    e_cnt = s2_ui.shape[0]
    assert e_cnt % _SS == 0
    nsuper = e_cnt // _SS
    nwb = n // _WB
    qmax = (nsuper + _NSUB - 1) // _NSUB

    @functools.partial(
        pl.kernel, mesh=_MESH, compiler_params=_SC_PARAMS,
        out_type=[jax.ShapeDtypeStruct((n, 128), jnp.float32)] * 2,
        scratch_types=[
            pltpu.VMEM((n,), jnp.float32),
            pltpu.VMEM((_SUB, _CB), jnp.int32),
            pltpu.VMEM((_SS,), jnp.int32),
            pltpu.VMEM((_SS,), jnp.int32),
            pltpu.VMEM((_SS,), jnp.float32),
            pltpu.VMEM((_CB, 128), jnp.float32),
            pltpu.VMEM((_CB, 128), jnp.float32),
            pltpu.VMEM((_WB, 128), jnp.float32),
            pltpu.VMEM_SHARED((n, 128), jnp.float32),
            pltpu.SemaphoreType.DMA,
        ])
    def agg_kernel(sx_ui_h, att_ui_h, dv_ui_h, s2_ui_h, ti_ui_h,
                   sx_iu_h, att_iu_h, dv_iu_h, s2_iu_h, ti_iu_h,
                   agg_ui_h, agg_iu_h, inv_tab, ti_b, ti_f, s2_f, w_b,
                   rows0, rows1, bounce, agg_sh, sem_g):
        c = lax.axis_index("c")
        s = lax.axis_index("s")

        def zrow(r, _):
            for j in range(8):
                bounce[r, pl.ds(16 * j, 16)] = jnp.zeros((16,), jnp.float32)
            return 0

        lax.fori_loop(0, _WB, zrow, 0)
        _for_each_wb_chunk(
            s, nwb, lambda r0: pltpu.sync_copy(
                bounce, agg_sh.at[pl.ds(r0, _WB)]))
        plsc.subcore_barrier()

        def process(sx_h, att_h, dv_h, s2_h, ti_h):
            pltpu.sync_copy(dv_h, inv_tab)

            def inv_body(j, _):
                sl = pl.ds(16 * j, 16)
                inv_tab[sl] = 1.0 / (inv_tab[sl] + 1e-06)
                return 0

            lax.fori_loop(0, n // 16, inv_body, 0)

            def on_buf(j, fn):
                @pl.when(j % 2 == 0)
                def _():
                    fn(rows0)

                @pl.when(j % 2 == 1)
                def _():
                    fn(rows1)

            def issue_gather(j):
                on_buf(j, lambda rb: pltpu.async_copy(
                    sx_h.at[s2_f.at[pl.ds(_CB * j, _CB)]], rb, sem_g))

            def wait_gather(j):
                on_buf(j, lambda rb: pltpu.make_async_copy(
                    sx_h.at[s2_f.at[pl.ds(_CB * j, _CB)]], rb, sem_g).wait())

            def scatter(j):
                on_buf(j, lambda rb: pltpu.sync_copy(
                    rb, agg_sh.at[ti_b.at[j]], add=True))

            def super_body(q, _):
                sc = s + _NSUB * q

                @pl.when(sc < nsuper)
                def _():
                    off = sc * _SS
                    pltpu.sync_copy(att_h.at[pl.ds(off, _SS)], w_b)
                    pltpu.sync_copy(s2_h.at[pl.ds(off, _SS)], s2_f)
                    pltpu.sync_copy(ti_h.at[pl.ds(off, _SS)], ti_f)

                    def wbody(i, _):
                        sl = pl.ds(16 * i, 16)
                        ti16 = ti_f[sl]
                        ti_b[i // 4, pl.ds(16 * (i % 4), 16)] = ti16
                        w_b[sl] = w_b[sl] * plsc.load_gather(
                            inv_tab, [ti16])
                        return 0

                    lax.fori_loop(0, _SS // 16, wbody, 0)
                    issue_gather(0)

                    def sub_body(j, _):
                        wait_gather(j)

                        @pl.when(j + 1 < _SUB)
                        def _():
                            issue_gather(j + 1)

                        def scale_in(rb):
                            def scale_body(e, _):
                                bvec = plsc.load_gather(
                                    w_b,
                                    [jnp.full((16,), _CB, jnp.int32) * j + e])
                                for r in range(8):
                                    sl = pl.ds(16 * r, 16)
                                    rb[e, sl] = rb[e, sl] * bvec
                                return 0

                            lax.fori_loop(0, _CB, scale_body, 0)

                        on_buf(j, scale_in)
                        scatter(j)
                        return 0

                    lax.fori_loop(0, _SUB, sub_body, 0)
                return 0

            lax.fori_loop(0, qmax, super_body, 0)

        @pl.when(c == 0)
        def _():
            process(sx_ui_h, att_ui_h, dv_ui_h, s2_ui_h, ti_ui_h)

        @pl.when(c == 1)
        def _():
            process(sx_iu_h, att_iu_h, dv_iu_h, s2_iu_h, ti_iu_h)

        plsc.subcore_barrier()

        def wb(r0):
            pltpu.sync_copy(agg_sh.at[pl.ds(r0, _WB)], bounce)

            @pl.when(c == 0)
            def _():
                pltpu.sync_copy(bounce, agg_ui_h.at[pl.ds(r0, _WB)])

            @pl.when(c == 1)
            def _():
                pltpu.sync_copy(bounce, agg_iu_h.at[pl.ds(r0, _WB)])

        _for_each_wb_chunk(s, nwb, wb)

    return agg_kernel(sx_ui, att_ui, dv_ui, s2_ui, ti_ui,
                      sx_iu, att_iu, dv_iu, s2_iu, ti_iu)


def _sc_conv_layer(zeros8, sx_ui, as_ui, at_ui, si_ui, ti_ui,
                   sx_iu, as_iu, at_iu, si_iu, ti_iu):
    s2_ui, att_ui, dv_ui, s2_iu, att_iu, dv_iu = _sc_att_layer(
        zeros8, as_ui, at_ui, si_ui, ti_ui, as_iu, at_iu, si_iu, ti_iu)
    return _sc_agg_layer(sx_ui, att_ui, dv_ui, s2_ui, ti_ui,
                         sx_iu, att_iu, dv_iu, s2_iu, ti_iu)


def kernel(x_user, x_item, edge_index_ui, edge_index_iu, params):
    n = x_user.shape[0]
    p = params
    si_ui, ti_ui = edge_index_ui[0], edge_index_ui[1]
    si_iu, ti_iu = edge_index_iu[0], edge_index_iu[1]

    w0 = [p["l0_ui_Wsrc"], p["l0_ui_Wdst"], p["l0_ui_a"],
          p["l0_iu_Wsrc"], p["l0_iu_Wdst"], p["l0_iu_a"]]
    (sx_ui, tx_ui, as_ui, at_ui, sx_iu, tx_iu, as_iu, at_iu) = (
        _run_layer_matmuls(_layer0_body, [x_user, x_item], w0, n))

    zeros8 = jnp.zeros((_C, 8), jnp.float32)
    agg_ui, agg_iu = _sc_conv_layer(
        zeros8, sx_ui, as_ui[:, 0], at_ui[:, 0], si_ui, ti_ui,
        sx_iu, as_iu[:, 0], at_iu[:, 0], si_iu, ti_iu)

    w1 = [p["l1_ui_Wsrc"], p["l1_ui_Wdst"], p["l1_ui_a"],
          p["l1_iu_Wsrc"], p["l1_iu_Wdst"], p["l1_iu_a"]]
    (sx_ui, tx_ui, as_ui, at_ui, sx_iu, tx_iu, as_iu, at_iu) = (
        _run_layer_matmuls(_layer1_body, [tx_ui, agg_ui, tx_iu, agg_iu],
                           w1, n))

    agg_ui, agg_iu = _sc_conv_layer(
        zeros8, sx_ui, as_ui[:, 0], at_ui[:, 0], si_ui, ti_ui,
        sx_iu, as_iu[:, 0], at_iu[:, 0], si_iu, ti_iu)

    wf = [p["lin_user_W"], p["lin_user_b"], p["lin_item_W"], p["lin_item_b"]]
    xu2, xi2, out_u, out_i = _run_final(
        [tx_ui, agg_ui, tx_iu, agg_iu], wf, n)
    return (xu2, xi2, out_u, out_i)


# speedup vs baseline: 10.0395x; 1.1923x over previous
import functools

import jax
import jax.numpy as jnp
from jax import lax
from jax.experimental import pallas as pl
from jax.experimental.pallas import tpu as pltpu
from jax.experimental.pallas import tpu_sc as plsc

_BLK = 2000


def _mm(a, b):
    return jax.lax.dot_general(
        a, b, (((1,), (0,)), ((), ())),
        precision=jax.lax.Precision.HIGHEST,
        preferred_element_type=jnp.float32,
    )


def _proj_pair(xs, xd, Wsrc, Wdst, a):
    sx = _mm(xs, Wsrc)
    tx = _mm(xd, Wdst)
    asrc = _mm(sx, a[:128])
    adst = _mm(tx, a[128:])
    B = sx.shape[0]
    return sx, tx, jnp.broadcast_to(asrc, (B, 8)), jnp.broadcast_to(adst, (B, 8))


def _layer0_body(xu_ref, xi_ref, wsu_ref, wdu_ref, au_ref, wsi_ref, wdi_ref,
                 ai_ref, sx_ui_ref, tx_ui_ref, as_ui_ref, at_ui_ref,
                 sx_iu_ref, tx_iu_ref, as_iu_ref, at_iu_ref):
    xu = xu_ref[...]
    xi = xi_ref[...]
    sx_ui_ref[...], tx_ui_ref[...], as_ui_ref[...], at_ui_ref[...] = (
        _proj_pair(xu, xi, wsu_ref[...], wdu_ref[...], au_ref[...]))
    sx_iu_ref[...], tx_iu_ref[...], as_iu_ref[...], at_iu_ref[...] = (
        _proj_pair(xi, xu, wsi_ref[...], wdi_ref[...], ai_ref[...]))


def _layer1_body(txp_ui_ref, agg_ui_ref, txp_iu_ref, agg_iu_ref, wsu_ref,
                 wdu_ref, au_ref, wsi_ref, wdi_ref, ai_ref, sx_ui_ref,
                 tx_ui_ref, as_ui_ref, at_ui_ref, sx_iu_ref, tx_iu_ref,
                 as_iu_ref, at_iu_ref):
    xi = jax.nn.relu(txp_ui_ref[...] + agg_ui_ref[...])
    xu = jax.nn.relu(txp_iu_ref[...] + agg_iu_ref[...])
    sx_ui_ref[...], tx_ui_ref[...], as_ui_ref[...], at_ui_ref[...] = (
        _proj_pair(xu, xi, wsu_ref[...], wdu_ref[...], au_ref[...]))
    sx_iu_ref[...], tx_iu_ref[...], as_iu_ref[...], at_iu_ref[...] = (
        _proj_pair(xi, xu, wsi_ref[...], wdi_ref[...], ai_ref[...]))


def _final_body(txp_ui_ref, agg_ui_ref, txp_iu_ref, agg_iu_ref, wu_ref,
                bu_ref, wi_ref, bi_ref, xu_ref, xi_ref, ou_ref, oi_ref):
    xi = jax.nn.relu(txp_ui_ref[...] + agg_ui_ref[...])
    xu = jax.nn.relu(txp_iu_ref[...] + agg_iu_ref[...])
    xu_ref[...] = xu
    xi_ref[...] = xi
    ou_ref[...] = _mm(xu, wu_ref[...]) + bu_ref[...]
    oi_ref[...] = _mm(xi, wi_ref[...]) + bi_ref[...]


def _row_spec(width):
    return pl.BlockSpec((_BLK, width), lambda i: (i, 0))


def _full_spec(shape):
    return pl.BlockSpec(shape, lambda i: tuple(0 for _ in shape))


def _run_layer_matmuls(body, row_inputs, weight_inputs, n):
    in_specs = [_row_spec(x.shape[1]) for x in row_inputs]
    in_specs += [_full_spec(w.shape) for w in weight_inputs]
    out_shape = [
        jax.ShapeDtypeStruct((n, 128), jnp.float32),
        jax.ShapeDtypeStruct((n, 128), jnp.float32),
        jax.ShapeDtypeStruct((n, 8), jnp.float32),
        jax.ShapeDtypeStruct((n, 8), jnp.float32),
    ] * 2
    out_specs = [_row_spec(s.shape[1]) for s in out_shape]
    return pl.pallas_call(
        body, grid=(n // _BLK,), in_specs=in_specs, out_specs=out_specs,
        out_shape=out_shape,
    )(*row_inputs, *weight_inputs)


def _run_final(row_inputs, weight_inputs, n):
    in_specs = [_row_spec(x.shape[1]) for x in row_inputs]
    in_specs += [_full_spec(w.shape) for w in weight_inputs]
    out_shape = [
        jax.ShapeDtypeStruct((n, 128), jnp.float32),
        jax.ShapeDtypeStruct((n, 128), jnp.float32),
        jax.ShapeDtypeStruct((n, 64), jnp.float32),
        jax.ShapeDtypeStruct((n, 64), jnp.float32),
    ]
    out_specs = [_row_spec(s.shape[1]) for s in out_shape]
    return pl.pallas_call(
        _final_body, grid=(n // _BLK,), in_specs=in_specs,
        out_specs=out_specs, out_shape=out_shape,
    )(*row_inputs, *weight_inputs)


_C = 128
_NSUB = 16
_WB = 80

_SC_PARAMS = pltpu.CompilerParams(
    needs_layout_passes=False, use_tc_tiling_on_sc=False)
_MESH = plsc.VectorSubcoreMesh(core_axis_name="c", subcore_axis_name="s")


def _for_each_wb_chunk(s, nwb, fn):
    def body(b, _):
        cid = s + _NSUB * b

        @pl.when(cid < nwb)
        def _():
            fn(cid * _WB)
        return 0

    lax.fori_loop(0, (nwb + _NSUB - 1) // _NSUB, body, 0)


_SSA = 1280


def _sc_att_layer(zeros8, as_ui, at_ui, si_ui, ti_ui, as_iu, at_iu, si_iu,
                  ti_iu):
    n = as_ui.shape[0]
    e_cnt = si_ui.shape[0]
    assert e_cnt % _SSA == 0 and n % _WB == 0
    nsuper = e_cnt // _SSA
    ngrp = _SSA // _C
    nwb = n // _WB
    qmax = (nsuper + _NSUB - 1) // _NSUB

    @functools.partial(
        pl.kernel, mesh=_MESH, compiler_params=_SC_PARAMS,
        out_type=[
            jax.ShapeDtypeStruct((e_cnt,), jnp.int32),
            jax.ShapeDtypeStruct((e_cnt,), jnp.float32),
            jax.ShapeDtypeStruct((n,), jnp.float32),
            jax.ShapeDtypeStruct((e_cnt,), jnp.int32),
            jax.ShapeDtypeStruct((e_cnt,), jnp.float32),
            jax.ShapeDtypeStruct((n,), jnp.float32),
        ],
        scratch_types=[
            pltpu.VMEM((n,), jnp.int32),
            pltpu.VMEM((n,), jnp.float32),
            pltpu.VMEM((n,), jnp.float32),
            pltpu.VMEM((_SSA,), jnp.int32),
            pltpu.VMEM((ngrp, _C), jnp.int32),
            pltpu.VMEM((_SSA,), jnp.int32),
            pltpu.VMEM((_SSA,), jnp.float32),
            pltpu.VMEM((_C, 8), jnp.float32),
            pltpu.VMEM((_WB, 8), jnp.float32),
            pltpu.VMEM((_WB,), jnp.float32),
            pltpu.VMEM_SHARED((n, 8), jnp.float32),
        ])
    def att_kernel(z8_h, as_ui_h, at_ui_h, si_ui_h, ti_ui_h, as_iu_h,
                   at_iu_h, si_iu_h, ti_iu_h, s2_ui_h, att_ui_h, dv_ui_h,
                   s2_iu_h, att_iu_h, dv_iu_h, si_tab, as_tab, at_tab, si_f,
                   ti_b, s2_f, att_f, att8, bounce, dvc, dv_sh):
        c = lax.axis_index("c")
        s = lax.axis_index("s")
        iota16 = lax.broadcasted_iota(jnp.int32, (16,), 0)

        pltpu.sync_copy(z8_h.at[pl.ds(0, _WB)], bounce)
        pltpu.sync_copy(z8_h, att8)
        _for_each_wb_chunk(
            s, nwb, lambda r0: pltpu.sync_copy(
                bounce, dv_sh.at[pl.ds(r0, _WB)]))
        plsc.subcore_barrier()

        def process(as_h, at_h, si_h, ti_h, s2_h, att_h):
            pltpu.sync_copy(si_h.at[pl.ds(0, n)], si_tab)
            pltpu.sync_copy(as_h, as_tab)
            pltpu.sync_copy(at_h, at_tab)

            def super_body(q, _):
                sc = s + _NSUB * q

                @pl.when(sc < nsuper)
                def _():
                    off = sc * _SSA
                    pltpu.sync_copy(si_h.at[pl.ds(off, _SSA)], si_f)
                    pltpu.sync_copy(ti_h.at[pl.ds(off, _SSA)], s2_f)

                    def vbody(i, _):
                        sl = pl.ds(16 * i, 16)
                        ti16 = s2_f[sl]
                        ti_b[i // 8, pl.ds(16 * (i % 8), 16)] = ti16
                        s2_16 = plsc.load_gather(si_tab, [si_f[sl]])
                        x = (plsc.load_gather(as_tab, [s2_16])
                             + plsc.load_gather(at_tab, [ti16]))
                        att16 = jnp.exp(jnp.where(x >= 0, x, 0.2 * x))
                        si_f[sl] = s2_16
                        att_f[sl] = att16
                        return 0

                    lax.fori_loop(0, _SSA // 16, vbody, 0)
                    pltpu.sync_copy(si_f, s2_h.at[pl.ds(off, _SSA)])
                    pltpu.sync_copy(att_f, att_h.at[pl.ds(off, _SSA)])

                    def grp_body(g, _):
                        def cpy(i, _):
                            e16 = iota16 + 16 * i
                            att16 = att_f[pl.ds(_C * g + 16 * i, 16)]
                            plsc.store_scatter(
                                att8, [e16, jnp.zeros((16,), jnp.int32)],
                                att16)
                            return 0

                        lax.fori_loop(0, _C // 16, cpy, 0)
                        pltpu.sync_copy(att8, dv_sh.at[ti_b.at[g]], add=True)
                        return 0

                    lax.fori_loop(0, ngrp, grp_body, 0)
                return 0

            lax.fori_loop(0, qmax, super_body, 0)

        @pl.when(c == 0)
        def _():
            process(as_ui_h, at_ui_h, si_ui_h, ti_ui_h, s2_ui_h, att_ui_h)

        @pl.when(c == 1)
        def _():
            process(as_iu_h, at_iu_h, si_iu_h, ti_iu_h, s2_iu_h, att_iu_h)

        plsc.subcore_barrier()

        def compact_out(r0):
            pltpu.sync_copy(dv_sh.at[pl.ds(r0, _WB)], bounce)
            for j in range(_WB // 16):
                e16 = iota16 + 16 * j
                dvc[pl.ds(16 * j, 16)] = plsc.load_gather(
                    bounce, [e16, jnp.zeros((16,), jnp.int32)])

            @pl.when(c == 0)
            def _():
                pltpu.sync_copy(dvc, dv_ui_h.at[pl.ds(r0, _WB)])

            @pl.when(c == 1)
            def _():
                pltpu.sync_copy(dvc, dv_iu_h.at[pl.ds(r0, _WB)])

        _for_each_wb_chunk(s, nwb, compact_out)

    return att_kernel(zeros8, as_ui, at_ui, si_ui, ti_ui,
                      as_iu, at_iu, si_iu, ti_iu)


_CB = 64
_SUB = 20
_SS = _SUB * _CB


def _sc_agg_layer(sx_ui, att_ui, dv_ui, s2_ui, ti_ui,
                  sx_iu, att_iu, dv_iu, s2_iu, ti_iu):
    n = sx_ui.shape[0]
    e_cnt = s2_ui.shape[0]
    assert e_cnt % _SS == 0
    nsuper = e_cnt // _SS
    nwb = n // _WB
    qmax = (nsuper + _NSUB - 1) // _NSUB

    @functools.partial(
        pl.kernel, mesh=_MESH, compiler_params=_SC_PARAMS,
        out_type=[jax.ShapeDtypeStruct((n, 128), jnp.float32)] * 2,
        scratch_types=[
            pltpu.VMEM((n,), jnp.float32),
            pltpu.VMEM((_SUB, _CB), jnp.int32),
            pltpu.VMEM((_SS,), jnp.int32),
            pltpu.VMEM((_SS,), jnp.int32),
            pltpu.VMEM((_SS,), jnp.float32),
            pltpu.VMEM((_CB, 128), jnp.float32),
            pltpu.VMEM((_CB, 128), jnp.float32),
            pltpu.VMEM((_WB, 128), jnp.float32),
            pltpu.VMEM_SHARED((n, 128), jnp.float32),
            pltpu.SemaphoreType.DMA,
            pltpu.SemaphoreType.DMA,
        ])
    def agg_kernel(sx_ui_h, att_ui_h, dv_ui_h, s2_ui_h, ti_ui_h,
                   sx_iu_h, att_iu_h, dv_iu_h, s2_iu_h, ti_iu_h,
                   agg_ui_h, agg_iu_h, inv_tab, ti_b, ti_f, s2_f, w_b,
                   rows0, rows1, bounce, agg_sh, sem_g, sem_c):
        c = lax.axis_index("c")
        s = lax.axis_index("s")

        def zrow(r, _):
            for j in range(8):
                bounce[r, pl.ds(16 * j, 16)] = jnp.zeros((16,), jnp.float32)
            return 0

        lax.fori_loop(0, _WB, zrow, 0)
        _for_each_wb_chunk(
            s, nwb, lambda r0: pltpu.sync_copy(
                bounce, agg_sh.at[pl.ds(r0, _WB)]))
        plsc.subcore_barrier()

        def process(sx_h, att_h, dv_h, s2_h, ti_h):
            pltpu.sync_copy(dv_h, inv_tab)

            def inv_body(j, _):
                sl = pl.ds(16 * j, 16)
                inv_tab[sl] = 1.0 / (inv_tab[sl] + 1e-06)
                return 0

            lax.fori_loop(0, n // 16, inv_body, 0)

            def on_buf(j, fn):
                @pl.when(j % 2 == 0)
                def _():
                    fn(rows0)

                @pl.when(j % 2 == 1)
                def _():
                    fn(rows1)

            def issue_gather(j):
                on_buf(j, lambda rb: pltpu.async_copy(
                    sx_h.at[s2_f.at[pl.ds(_CB * j, _CB)]], rb, sem_g))

            def wait_gather(j):
                on_buf(j, lambda rb: pltpu.make_async_copy(
                    sx_h.at[s2_f.at[pl.ds(_CB * j, _CB)]], rb, sem_g).wait())

            def issue_scatter(j):
                on_buf(j, lambda rb: pltpu.async_copy(
                    rb, agg_sh.at[ti_b.at[j]], sem_c, add=True))

            def wait_scatter(j):
                on_buf(j, lambda rb: pltpu.make_async_copy(
                    rb, agg_sh.at[ti_b.at[j]], sem_c).wait())

            def super_body(q, _):
                sc = s + _NSUB * q

                @pl.when(sc < nsuper)
                def _():
                    off = sc * _SS
                    pltpu.sync_copy(att_h.at[pl.ds(off, _SS)], w_b)
                    pltpu.sync_copy(s2_h.at[pl.ds(off, _SS)], s2_f)
                    pltpu.sync_copy(ti_h.at[pl.ds(off, _SS)], ti_f)

                    def wbody(i, _):
                        sl = pl.ds(16 * i, 16)
                        ti16 = ti_f[sl]
                        ti_b[i // 4, pl.ds(16 * (i % 4), 16)] = ti16
                        w_b[sl] = w_b[sl] * plsc.load_gather(
                            inv_tab, [ti16])
                        return 0

                    lax.fori_loop(0, _SS // 16, wbody, 0)
                    issue_gather(0)

                    def sub_body(j, _):
                        wait_gather(j)

                        @pl.when(j + 1 < _SUB)
                        def _():
                            @pl.when(j >= 1)
                            def _():
                                wait_scatter(j - 1)
                            issue_gather(j + 1)

                        def scale_in(rb):
                            def scale_body(e, _):
                                bvec = plsc.load_gather(
                                    w_b,
                                    [jnp.full((16,), _CB, jnp.int32) * j + e])
                                for r in range(8):
                                    sl = pl.ds(16 * r, 16)
                                    rb[e, sl] = rb[e, sl] * bvec
                                return 0

                            lax.fori_loop(0, _CB, scale_body, 0)

                        on_buf(j, scale_in)
                        issue_scatter(j)
                        return 0

                    lax.fori_loop(0, _SUB, sub_body, 0)
                    wait_scatter(_SUB - 2)
                    wait_scatter(_SUB - 1)
                return 0

            lax.fori_loop(0, qmax, super_body, 0)

        @pl.when(c == 0)
        def _():
            process(sx_ui_h, att_ui_h, dv_ui_h, s2_ui_h, ti_ui_h)

        @pl.when(c == 1)
        def _():
            process(sx_iu_h, att_iu_h, dv_iu_h, s2_iu_h, ti_iu_h)

        plsc.subcore_barrier()

        def wb(r0):
            pltpu.sync_copy(agg_sh.at[pl.ds(r0, _WB)], bounce)

            @pl.when(c == 0)
            def _():
                pltpu.sync_copy(bounce, agg_ui_h.at[pl.ds(r0, _WB)])

            @pl.when(c == 1)
            def _():
                pltpu.sync_copy(bounce, agg_iu_h.at[pl.ds(r0, _WB)])

        _for_each_wb_chunk(s, nwb, wb)

    return agg_kernel(sx_ui, att_ui, dv_ui, s2_ui, ti_ui,
                      sx_iu, att_iu, dv_iu, s2_iu, ti_iu)


def _sc_conv_layer(zeros8, sx_ui, as_ui, at_ui, si_ui, ti_ui,
                   sx_iu, as_iu, at_iu, si_iu, ti_iu):
    s2_ui, att_ui, dv_ui, s2_iu, att_iu, dv_iu = _sc_att_layer(
        zeros8, as_ui, at_ui, si_ui, ti_ui, as_iu, at_iu, si_iu, ti_iu)
    return _sc_agg_layer(sx_ui, att_ui, dv_ui, s2_ui, ti_ui,
                         sx_iu, att_iu, dv_iu, s2_iu, ti_iu)


def kernel(x_user, x_item, edge_index_ui, edge_index_iu, params):
    n = x_user.shape[0]
    p = params
    si_ui, ti_ui = edge_index_ui[0], edge_index_ui[1]
    si_iu, ti_iu = edge_index_iu[0], edge_index_iu[1]

    w0 = [p["l0_ui_Wsrc"], p["l0_ui_Wdst"], p["l0_ui_a"],
          p["l0_iu_Wsrc"], p["l0_iu_Wdst"], p["l0_iu_a"]]
    (sx_ui, tx_ui, as_ui, at_ui, sx_iu, tx_iu, as_iu, at_iu) = (
        _run_layer_matmuls(_layer0_body, [x_user, x_item], w0, n))

    zeros8 = jnp.zeros((_C, 8), jnp.float32)
    agg_ui, agg_iu = _sc_conv_layer(
        zeros8, sx_ui, as_ui[:, 0], at_ui[:, 0], si_ui, ti_ui,
        sx_iu, as_iu[:, 0], at_iu[:, 0], si_iu, ti_iu)

    w1 = [p["l1_ui_Wsrc"], p["l1_ui_Wdst"], p["l1_ui_a"],
          p["l1_iu_Wsrc"], p["l1_iu_Wdst"], p["l1_iu_a"]]
    (sx_ui, tx_ui, as_ui, at_ui, sx_iu, tx_iu, as_iu, at_iu) = (
        _run_layer_matmuls(_layer1_body, [tx_ui, agg_ui, tx_iu, agg_iu],
                           w1, n))

    agg_ui, agg_iu = _sc_conv_layer(
        zeros8, sx_ui, as_ui[:, 0], at_ui[:, 0], si_ui, ti_ui,
        sx_iu, as_iu[:, 0], at_iu[:, 0], si_iu, ti_iu)

    wf = [p["lin_user_W"], p["lin_user_b"], p["lin_item_W"], p["lin_item_b"]]
    xu2, xi2, out_u, out_i = _run_final(
        [tx_ui, agg_ui, tx_iu, agg_iu], wf, n)
    return (xu2, xi2, out_u, out_i)


# speedup vs baseline: 10.6998x; 1.0658x over previous
import functools

import jax
import jax.numpy as jnp
from jax import lax
from jax.experimental import pallas as pl
from jax.experimental.pallas import tpu as pltpu
from jax.experimental.pallas import tpu_sc as plsc

_BLK = 2000


def _mm(a, b):
    return jax.lax.dot_general(
        a, b, (((1,), (0,)), ((), ())),
        precision=jax.lax.Precision.HIGHEST,
        preferred_element_type=jnp.float32,
    )


def _proj_pair(xs, xd, Wsrc, Wdst, a):
    sx = _mm(xs, Wsrc)
    tx = _mm(xd, Wdst)
    asrc = _mm(sx, a[:128])
    adst = _mm(tx, a[128:])
    B = sx.shape[0]
    return sx, tx, jnp.broadcast_to(asrc, (B, 8)), jnp.broadcast_to(adst, (B, 8))


def _layer0_body(xu_ref, xi_ref, wsu_ref, wdu_ref, au_ref, wsi_ref, wdi_ref,
                 ai_ref, sx_ui_ref, tx_ui_ref, as_ui_ref, at_ui_ref,
                 sx_iu_ref, tx_iu_ref, as_iu_ref, at_iu_ref):
    xu = xu_ref[...]
    xi = xi_ref[...]
    sx_ui_ref[...], tx_ui_ref[...], as_ui_ref[...], at_ui_ref[...] = (
        _proj_pair(xu, xi, wsu_ref[...], wdu_ref[...], au_ref[...]))
    sx_iu_ref[...], tx_iu_ref[...], as_iu_ref[...], at_iu_ref[...] = (
        _proj_pair(xi, xu, wsi_ref[...], wdi_ref[...], ai_ref[...]))


def _layer1_body(txp_ui_ref, agg_ui_ref, txp_iu_ref, agg_iu_ref, wsu_ref,
                 wdu_ref, au_ref, wsi_ref, wdi_ref, ai_ref, sx_ui_ref,
                 tx_ui_ref, as_ui_ref, at_ui_ref, sx_iu_ref, tx_iu_ref,
                 as_iu_ref, at_iu_ref):
    xi = jax.nn.relu(txp_ui_ref[...] + agg_ui_ref[...])
    xu = jax.nn.relu(txp_iu_ref[...] + agg_iu_ref[...])
    sx_ui_ref[...], tx_ui_ref[...], as_ui_ref[...], at_ui_ref[...] = (
        _proj_pair(xu, xi, wsu_ref[...], wdu_ref[...], au_ref[...]))
    sx_iu_ref[...], tx_iu_ref[...], as_iu_ref[...], at_iu_ref[...] = (
        _proj_pair(xi, xu, wsi_ref[...], wdi_ref[...], ai_ref[...]))


def _final_body(txp_ui_ref, agg_ui_ref, txp_iu_ref, agg_iu_ref, wu_ref,
                bu_ref, wi_ref, bi_ref, xu_ref, xi_ref, ou_ref, oi_ref):
    xi = jax.nn.relu(txp_ui_ref[...] + agg_ui_ref[...])
    xu = jax.nn.relu(txp_iu_ref[...] + agg_iu_ref[...])
    xu_ref[...] = xu
    xi_ref[...] = xi
    ou_ref[...] = _mm(xu, wu_ref[...]) + bu_ref[...]
    oi_ref[...] = _mm(xi, wi_ref[...]) + bi_ref[...]


def _row_spec(width):
    return pl.BlockSpec((_BLK, width), lambda i: (i, 0))


def _full_spec(shape):
    return pl.BlockSpec(shape, lambda i: tuple(0 for _ in shape))


def _run_layer_matmuls(body, row_inputs, weight_inputs, n):
    in_specs = [_row_spec(x.shape[1]) for x in row_inputs]
    in_specs += [_full_spec(w.shape) for w in weight_inputs]
    out_shape = [
        jax.ShapeDtypeStruct((n, 128), jnp.float32),
        jax.ShapeDtypeStruct((n, 128), jnp.float32),
        jax.ShapeDtypeStruct((n, 8), jnp.float32),
        jax.ShapeDtypeStruct((n, 8), jnp.float32),
    ] * 2
    out_specs = [_row_spec(s.shape[1]) for s in out_shape]
    return pl.pallas_call(
        body, grid=(n // _BLK,), in_specs=in_specs, out_specs=out_specs,
        out_shape=out_shape,
    )(*row_inputs, *weight_inputs)


def _run_final(row_inputs, weight_inputs, n):
    in_specs = [_row_spec(x.shape[1]) for x in row_inputs]
    in_specs += [_full_spec(w.shape) for w in weight_inputs]
    out_shape = [
        jax.ShapeDtypeStruct((n, 128), jnp.float32),
        jax.ShapeDtypeStruct((n, 128), jnp.float32),
        jax.ShapeDtypeStruct((n, 64), jnp.float32),
        jax.ShapeDtypeStruct((n, 64), jnp.float32),
    ]
    out_specs = [_row_spec(s.shape[1]) for s in out_shape]
    return pl.pallas_call(
        _final_body, grid=(n // _BLK,), in_specs=in_specs,
        out_specs=out_specs, out_shape=out_shape,
    )(*row_inputs, *weight_inputs)


_C = 128
_NSUB = 16
_WB = 80

_SC_PARAMS = pltpu.CompilerParams(
    needs_layout_passes=False, use_tc_tiling_on_sc=False)
_MESH = plsc.VectorSubcoreMesh(core_axis_name="c", subcore_axis_name="s")


def _for_each_wb_chunk(s, nwb, fn):
    def body(b, _):
        cid = s + _NSUB * b

        @pl.when(cid < nwb)
        def _():
            fn(cid * _WB)
        return 0

    lax.fori_loop(0, (nwb + _NSUB - 1) // _NSUB, body, 0)


_SSA = 1280


def _sc_att_layer(zeros8, as_ui, at_ui, si_ui, ti_ui, as_iu, at_iu, si_iu,
                  ti_iu):
    n = as_ui.shape[0]
    e_cnt = si_ui.shape[0]
    assert e_cnt % _SSA == 0 and n % _WB == 0
    nsuper = e_cnt // _SSA
    ngrp = _SSA // _C
    nwb = n // _WB
    qmax = (nsuper + _NSUB - 1) // _NSUB

    @functools.partial(
        pl.kernel, mesh=_MESH, compiler_params=_SC_PARAMS,
        out_type=[
            jax.ShapeDtypeStruct((e_cnt,), jnp.int32),
            jax.ShapeDtypeStruct((e_cnt,), jnp.float32),
            jax.ShapeDtypeStruct((n,), jnp.float32),
            jax.ShapeDtypeStruct((e_cnt,), jnp.int32),
            jax.ShapeDtypeStruct((e_cnt,), jnp.float32),
            jax.ShapeDtypeStruct((n,), jnp.float32),
        ],
        scratch_types=[
            pltpu.VMEM((n,), jnp.int32),
            pltpu.VMEM((n,), jnp.float32),
            pltpu.VMEM((n,), jnp.float32),
            pltpu.VMEM((_SSA,), jnp.int32),
            pltpu.VMEM((ngrp, _C), jnp.int32),
            pltpu.VMEM((_SSA,), jnp.int32),
            pltpu.VMEM((_SSA,), jnp.float32),
            pltpu.VMEM((_C, 8), jnp.float32),
            pltpu.VMEM((_WB, 8), jnp.float32),
            pltpu.VMEM((_WB,), jnp.float32),
            pltpu.VMEM_SHARED((n, 8), jnp.float32),
        ])
    def att_kernel(z8_h, as_ui_h, at_ui_h, si_ui_h, ti_ui_h, as_iu_h,
                   at_iu_h, si_iu_h, ti_iu_h, s2_ui_h, att_ui_h, dv_ui_h,
                   s2_iu_h, att_iu_h, dv_iu_h, si_tab, as_tab, at_tab, si_f,
                   ti_b, s2_f, att_f, att8, bounce, dvc, dv_sh):
        c = lax.axis_index("c")
        s = lax.axis_index("s")
        iota16 = lax.broadcasted_iota(jnp.int32, (16,), 0)

        pltpu.sync_copy(z8_h.at[pl.ds(0, _WB)], bounce)
        pltpu.sync_copy(z8_h, att8)
        _for_each_wb_chunk(
            s, nwb, lambda r0: pltpu.sync_copy(
                bounce, dv_sh.at[pl.ds(r0, _WB)]))
        plsc.subcore_barrier()

        def process(as_h, at_h, si_h, ti_h, s2_h, att_h):
            pltpu.sync_copy(si_h.at[pl.ds(0, n)], si_tab)
            pltpu.sync_copy(as_h, as_tab)
            pltpu.sync_copy(at_h, at_tab)

            def super_body(q, _):
                sc = s + _NSUB * q

                @pl.when(sc < nsuper)
                def _():
                    off = sc * _SSA
                    pltpu.sync_copy(si_h.at[pl.ds(off, _SSA)], si_f)
                    pltpu.sync_copy(ti_h.at[pl.ds(off, _SSA)], s2_f)

                    @plsc.parallel_loop(0, _SSA // 16, unroll=2)
                    def _(i):
                        sl = pl.ds(16 * i, 16)
                        ti16 = s2_f[sl]
                        ti_b[i // 8, pl.ds(16 * (i % 8), 16)] = ti16
                        s2_16 = plsc.load_gather(si_tab, [si_f[sl]])
                        x = (plsc.load_gather(as_tab, [s2_16])
                             + plsc.load_gather(at_tab, [ti16]))
                        att16 = jnp.exp(jnp.where(x >= 0, x, 0.2 * x))
                        si_f[sl] = s2_16
                        att_f[sl] = att16
                    pltpu.sync_copy(si_f, s2_h.at[pl.ds(off, _SSA)])
                    pltpu.sync_copy(att_f, att_h.at[pl.ds(off, _SSA)])

                    def grp_body(g, _):
                        def cpy(i, _):
                            e16 = iota16 + 16 * i
                            att16 = att_f[pl.ds(_C * g + 16 * i, 16)]
                            plsc.store_scatter(
                                att8, [e16, jnp.zeros((16,), jnp.int32)],
                                att16)
                            return 0

                        lax.fori_loop(0, _C // 16, cpy, 0)
                        pltpu.sync_copy(att8, dv_sh.at[ti_b.at[g]], add=True)
                        return 0

                    lax.fori_loop(0, ngrp, grp_body, 0)
                return 0

            lax.fori_loop(0, qmax, super_body, 0)

        @pl.when(c == 0)
        def _():
            process(as_ui_h, at_ui_h, si_ui_h, ti_ui_h, s2_ui_h, att_ui_h)

        @pl.when(c == 1)
        def _():
            process(as_iu_h, at_iu_h, si_iu_h, ti_iu_h, s2_iu_h, att_iu_h)

        plsc.subcore_barrier()

        def compact_out(r0):
            pltpu.sync_copy(dv_sh.at[pl.ds(r0, _WB)], bounce)
            for j in range(_WB // 16):
                e16 = iota16 + 16 * j
                dvc[pl.ds(16 * j, 16)] = plsc.load_gather(
                    bounce, [e16, jnp.zeros((16,), jnp.int32)])

            @pl.when(c == 0)
            def _():
                pltpu.sync_copy(dvc, dv_ui_h.at[pl.ds(r0, _WB)])

            @pl.when(c == 1)
            def _():
                pltpu.sync_copy(dvc, dv_iu_h.at[pl.ds(r0, _WB)])

        _for_each_wb_chunk(s, nwb, compact_out)

    return att_kernel(zeros8, as_ui, at_ui, si_ui, ti_ui,
                      as_iu, at_iu, si_iu, ti_iu)


_CB = 64
_SUB = 20
_SS = _SUB * _CB


def _sc_agg_layer(sx_ui, att_ui, dv_ui, s2_ui, ti_ui,
                  sx_iu, att_iu, dv_iu, s2_iu, ti_iu):
    n = sx_ui.shape[0]
    e_cnt = s2_ui.shape[0]
    assert e_cnt % _SS == 0
    nsuper = e_cnt // _SS
    nwb = n // _WB
    qmax = (nsuper + _NSUB - 1) // _NSUB

    @functools.partial(
        pl.kernel, mesh=_MESH, compiler_params=_SC_PARAMS,
        out_type=[jax.ShapeDtypeStruct((n, 128), jnp.float32)] * 2,
        scratch_types=[
            pltpu.VMEM((n,), jnp.float32),
            pltpu.VMEM((_SUB, _CB), jnp.int32),
            pltpu.VMEM((_SS,), jnp.int32),
            pltpu.VMEM((_SS,), jnp.int32),
            pltpu.VMEM((_SS,), jnp.float32),
            pltpu.VMEM((_CB, 128), jnp.float32),
            pltpu.VMEM((_CB, 128), jnp.float32),
            pltpu.VMEM((_WB, 128), jnp.float32),
            pltpu.VMEM_SHARED((n, 128), jnp.float32),
            pltpu.SemaphoreType.DMA,
            pltpu.SemaphoreType.DMA,
        ])
    def agg_kernel(sx_ui_h, att_ui_h, dv_ui_h, s2_ui_h, ti_ui_h,
                   sx_iu_h, att_iu_h, dv_iu_h, s2_iu_h, ti_iu_h,
                   agg_ui_h, agg_iu_h, inv_tab, ti_b, ti_f, s2_f, w_b,
                   rows0, rows1, bounce, agg_sh, sem_g, sem_c):
        c = lax.axis_index("c")
        s = lax.axis_index("s")

        def zrow(r, _):
            for j in range(8):
                bounce[r, pl.ds(16 * j, 16)] = jnp.zeros((16,), jnp.float32)
            return 0

        lax.fori_loop(0, _WB, zrow, 0)
        _for_each_wb_chunk(
            s, nwb, lambda r0: pltpu.sync_copy(
                bounce, agg_sh.at[pl.ds(r0, _WB)]))
        plsc.subcore_barrier()

        def process(sx_h, att_h, dv_h, s2_h, ti_h):
            pltpu.sync_copy(dv_h, inv_tab)

            def inv_body(j, _):
                sl = pl.ds(16 * j, 16)
                inv_tab[sl] = 1.0 / (inv_tab[sl] + 1e-06)
                return 0

            lax.fori_loop(0, n // 16, inv_body, 0)

            def on_buf(j, fn):
                @pl.when(j % 2 == 0)
                def _():
                    fn(rows0)

                @pl.when(j % 2 == 1)
                def _():
                    fn(rows1)

            def issue_gather(j):
                on_buf(j, lambda rb: pltpu.async_copy(
                    sx_h.at[s2_f.at[pl.ds(_CB * j, _CB)]], rb, sem_g))

            def wait_gather(j):
                on_buf(j, lambda rb: pltpu.make_async_copy(
                    sx_h.at[s2_f.at[pl.ds(_CB * j, _CB)]], rb, sem_g).wait())

            def issue_scatter(j):
                on_buf(j, lambda rb: pltpu.async_copy(
                    rb, agg_sh.at[ti_b.at[j]], sem_c, add=True))

            def wait_scatter(j):
                on_buf(j, lambda rb: pltpu.make_async_copy(
                    rb, agg_sh.at[ti_b.at[j]], sem_c).wait())

            def super_body(q, _):
                sc = s + _NSUB * q

                @pl.when(sc < nsuper)
                def _():
                    off = sc * _SS
                    pltpu.sync_copy(att_h.at[pl.ds(off, _SS)], w_b)
                    pltpu.sync_copy(s2_h.at[pl.ds(off, _SS)], s2_f)
                    pltpu.sync_copy(ti_h.at[pl.ds(off, _SS)], ti_f)

                    @plsc.parallel_loop(0, _SS // 16, unroll=2)
                    def _(i):
                        sl = pl.ds(16 * i, 16)
                        ti16 = ti_f[sl]
                        ti_b[i // 4, pl.ds(16 * (i % 4), 16)] = ti16
                        w_b[sl] = w_b[sl] * plsc.load_gather(
                            inv_tab, [ti16])

                    issue_gather(0)

                    def sub_body(j, _):
                        wait_gather(j)

                        @pl.when(j + 1 < _SUB)
                        def _():
                            @pl.when(j >= 1)
                            def _():
                                wait_scatter(j - 1)
                            issue_gather(j + 1)

                        def scale_in(rb):
                            @plsc.parallel_loop(0, _CB, unroll=4)
                            def _(e):
                                bvec = plsc.load_gather(
                                    w_b,
                                    [jnp.full((16,), _CB, jnp.int32) * j + e])
                                for r in range(8):
                                    sl = pl.ds(16 * r, 16)
                                    rb[e, sl] = rb[e, sl] * bvec

                        on_buf(j, scale_in)
                        issue_scatter(j)
                        return 0

                    lax.fori_loop(0, _SUB, sub_body, 0)
                    wait_scatter(_SUB - 2)
                    wait_scatter(_SUB - 1)
                return 0

            lax.fori_loop(0, qmax, super_body, 0)

        @pl.when(c == 0)
        def _():
            process(sx_ui_h, att_ui_h, dv_ui_h, s2_ui_h, ti_ui_h)

        @pl.when(c == 1)
        def _():
            process(sx_iu_h, att_iu_h, dv_iu_h, s2_iu_h, ti_iu_h)

        plsc.subcore_barrier()

        def wb(r0):
            pltpu.sync_copy(agg_sh.at[pl.ds(r0, _WB)], bounce)

            @pl.when(c == 0)
            def _():
                pltpu.sync_copy(bounce, agg_ui_h.at[pl.ds(r0, _WB)])

            @pl.when(c == 1)
            def _():
                pltpu.sync_copy(bounce, agg_iu_h.at[pl.ds(r0, _WB)])

        _for_each_wb_chunk(s, nwb, wb)

    return agg_kernel(sx_ui, att_ui, dv_ui, s2_ui, ti_ui,
                      sx_iu, att_iu, dv_iu, s2_iu, ti_iu)


def _sc_conv_layer(zeros8, sx_ui, as_ui, at_ui, si_ui, ti_ui,
                   sx_iu, as_iu, at_iu, si_iu, ti_iu):
    s2_ui, att_ui, dv_ui, s2_iu, att_iu, dv_iu = _sc_att_layer(
        zeros8, as_ui, at_ui, si_ui, ti_ui, as_iu, at_iu, si_iu, ti_iu)
    return _sc_agg_layer(sx_ui, att_ui, dv_ui, s2_ui, ti_ui,
                         sx_iu, att_iu, dv_iu, s2_iu, ti_iu)


def kernel(x_user, x_item, edge_index_ui, edge_index_iu, params):
    n = x_user.shape[0]
    p = params
    si_ui, ti_ui = edge_index_ui[0], edge_index_ui[1]
    si_iu, ti_iu = edge_index_iu[0], edge_index_iu[1]

    w0 = [p["l0_ui_Wsrc"], p["l0_ui_Wdst"], p["l0_ui_a"],
          p["l0_iu_Wsrc"], p["l0_iu_Wdst"], p["l0_iu_a"]]
    (sx_ui, tx_ui, as_ui, at_ui, sx_iu, tx_iu, as_iu, at_iu) = (
        _run_layer_matmuls(_layer0_body, [x_user, x_item], w0, n))

    zeros8 = jnp.zeros((_C, 8), jnp.float32)
    agg_ui, agg_iu = _sc_conv_layer(
        zeros8, sx_ui, as_ui[:, 0], at_ui[:, 0], si_ui, ti_ui,
        sx_iu, as_iu[:, 0], at_iu[:, 0], si_iu, ti_iu)

    w1 = [p["l1_ui_Wsrc"], p["l1_ui_Wdst"], p["l1_ui_a"],
          p["l1_iu_Wsrc"], p["l1_iu_Wdst"], p["l1_iu_a"]]
    (sx_ui, tx_ui, as_ui, at_ui, sx_iu, tx_iu, as_iu, at_iu) = (
        _run_layer_matmuls(_layer1_body, [tx_ui, agg_ui, tx_iu, agg_iu],
                           w1, n))

    agg_ui, agg_iu = _sc_conv_layer(
        zeros8, sx_ui, as_ui[:, 0], at_ui[:, 0], si_ui, ti_ui,
        sx_iu, as_iu[:, 0], at_iu[:, 0], si_iu, ti_iu)

    wf = [p["lin_user_W"], p["lin_user_b"], p["lin_item_W"], p["lin_item_b"]]
    xu2, xi2, out_u, out_i = _run_final(
        [tx_ui, agg_ui, tx_iu, agg_iu], wf, n)
    return (xu2, xi2, out_u, out_i)


# speedup vs baseline: 12.1009x; 1.1310x over previous
import functools

import jax
import jax.numpy as jnp
from jax import lax
from jax.experimental import pallas as pl
from jax.experimental.pallas import tpu as pltpu
from jax.experimental.pallas import tpu_sc as plsc

_BLK = 2000


def _mm(a, b):
    return jax.lax.dot_general(
        a, b, (((1,), (0,)), ((), ())),
        precision=jax.lax.Precision.HIGHEST,
        preferred_element_type=jnp.float32,
    )


def _proj_pair(xs, xd, Wsrc, Wdst, a):
    sx = _mm(xs, Wsrc)
    tx = _mm(xd, Wdst)
    asrc = _mm(sx, a[:128])
    adst = _mm(tx, a[128:])
    B = sx.shape[0]
    return sx, tx, jnp.broadcast_to(asrc, (B, 8)), jnp.broadcast_to(adst, (B, 8))


def _layer0_body(xu_ref, xi_ref, wsu_ref, wdu_ref, au_ref, wsi_ref, wdi_ref,
                 ai_ref, sx_ui_ref, tx_ui_ref, as_ui_ref, at_ui_ref,
                 sx_iu_ref, tx_iu_ref, as_iu_ref, at_iu_ref):
    xu = xu_ref[...]
    xi = xi_ref[...]
    sx_ui_ref[...], tx_ui_ref[...], as_ui_ref[...], at_ui_ref[...] = (
        _proj_pair(xu, xi, wsu_ref[...], wdu_ref[...], au_ref[...]))
    sx_iu_ref[...], tx_iu_ref[...], as_iu_ref[...], at_iu_ref[...] = (
        _proj_pair(xi, xu, wsi_ref[...], wdi_ref[...], ai_ref[...]))


def _layer1_body(txp_ui_ref, agg_ui_ref, txp_iu_ref, agg_iu_ref, wsu_ref,
                 wdu_ref, au_ref, wsi_ref, wdi_ref, ai_ref, sx_ui_ref,
                 tx_ui_ref, as_ui_ref, at_ui_ref, sx_iu_ref, tx_iu_ref,
                 as_iu_ref, at_iu_ref):
    xi = jax.nn.relu(txp_ui_ref[...] + agg_ui_ref[...])
    xu = jax.nn.relu(txp_iu_ref[...] + agg_iu_ref[...])
    sx_ui_ref[...], tx_ui_ref[...], as_ui_ref[...], at_ui_ref[...] = (
        _proj_pair(xu, xi, wsu_ref[...], wdu_ref[...], au_ref[...]))
    sx_iu_ref[...], tx_iu_ref[...], as_iu_ref[...], at_iu_ref[...] = (
        _proj_pair(xi, xu, wsi_ref[...], wdi_ref[...], ai_ref[...]))


def _final_body(txp_ui_ref, agg_ui_ref, txp_iu_ref, agg_iu_ref, wu_ref,
                bu_ref, wi_ref, bi_ref, xu_ref, xi_ref, ou_ref, oi_ref):
    xi = jax.nn.relu(txp_ui_ref[...] + agg_ui_ref[...])
    xu = jax.nn.relu(txp_iu_ref[...] + agg_iu_ref[...])
    xu_ref[...] = xu
    xi_ref[...] = xi
    ou_ref[...] = _mm(xu, wu_ref[...]) + bu_ref[...]
    oi_ref[...] = _mm(xi, wi_ref[...]) + bi_ref[...]


def _row_spec(width):
    return pl.BlockSpec((_BLK, width), lambda i: (i, 0))


def _full_spec(shape):
    return pl.BlockSpec(shape, lambda i: tuple(0 for _ in shape))


def _run_layer_matmuls(body, row_inputs, weight_inputs, n):
    in_specs = [_row_spec(x.shape[1]) for x in row_inputs]
    in_specs += [_full_spec(w.shape) for w in weight_inputs]
    out_shape = [
        jax.ShapeDtypeStruct((n, 128), jnp.float32),
        jax.ShapeDtypeStruct((n, 128), jnp.float32),
        jax.ShapeDtypeStruct((n, 8), jnp.float32),
        jax.ShapeDtypeStruct((n, 8), jnp.float32),
    ] * 2
    out_specs = [_row_spec(s.shape[1]) for s in out_shape]
    return pl.pallas_call(
        body, grid=(n // _BLK,), in_specs=in_specs, out_specs=out_specs,
        out_shape=out_shape,
    )(*row_inputs, *weight_inputs)


def _run_final(row_inputs, weight_inputs, n):
    in_specs = [_row_spec(x.shape[1]) for x in row_inputs]
    in_specs += [_full_spec(w.shape) for w in weight_inputs]
    out_shape = [
        jax.ShapeDtypeStruct((n, 128), jnp.float32),
        jax.ShapeDtypeStruct((n, 128), jnp.float32),
        jax.ShapeDtypeStruct((n, 64), jnp.float32),
        jax.ShapeDtypeStruct((n, 64), jnp.float32),
    ]
    out_specs = [_row_spec(s.shape[1]) for s in out_shape]
    return pl.pallas_call(
        _final_body, grid=(n // _BLK,), in_specs=in_specs,
        out_specs=out_specs, out_shape=out_shape,
    )(*row_inputs, *weight_inputs)


_C = 128
_NSUB = 16
_WB = 80

_SC_PARAMS = pltpu.CompilerParams(
    needs_layout_passes=False, use_tc_tiling_on_sc=False)
_MESH = plsc.VectorSubcoreMesh(core_axis_name="c", subcore_axis_name="s")


def _for_each_wb_chunk(s, nwb, fn):
    def body(b, _):
        cid = s + _NSUB * b

        @pl.when(cid < nwb)
        def _():
            fn(cid * _WB)
        return 0

    lax.fori_loop(0, (nwb + _NSUB - 1) // _NSUB, body, 0)


_SSA = 1280


def _sc_att_layer(zeros8, as_ui, at_ui, si_ui, ti_ui, as_iu, at_iu, si_iu,
                  ti_iu):
    n = as_ui.shape[0]
    e_cnt = si_ui.shape[0]
    assert e_cnt % _SSA == 0 and n % _WB == 0
    nsuper = e_cnt // _SSA
    ngrp = _SSA // _C
    nwb = n // _WB
    qmax = (nsuper + _NSUB - 1) // _NSUB

    @functools.partial(
        pl.kernel, mesh=_MESH, compiler_params=_SC_PARAMS,
        out_type=[
            jax.ShapeDtypeStruct((e_cnt,), jnp.int32),
            jax.ShapeDtypeStruct((e_cnt,), jnp.float32),
            jax.ShapeDtypeStruct((n,), jnp.float32),
            jax.ShapeDtypeStruct((e_cnt,), jnp.int32),
            jax.ShapeDtypeStruct((e_cnt,), jnp.float32),
            jax.ShapeDtypeStruct((n,), jnp.float32),
        ],
        scratch_types=[
            pltpu.VMEM((n,), jnp.int32),
            pltpu.VMEM((n,), jnp.float32),
            pltpu.VMEM((n,), jnp.float32),
            pltpu.VMEM((_SSA,), jnp.int32),
            pltpu.VMEM((ngrp, _C), jnp.int32),
            pltpu.VMEM((_SSA,), jnp.int32),
            pltpu.VMEM((_SSA,), jnp.float32),
            pltpu.VMEM((_C, 8), jnp.float32),
            pltpu.VMEM((_WB, 8), jnp.float32),
            pltpu.VMEM((_WB,), jnp.float32),
            pltpu.VMEM_SHARED((n, 8), jnp.float32),
        ])
    def att_kernel(z8_h, as_ui_h, at_ui_h, si_ui_h, ti_ui_h, as_iu_h,
                   at_iu_h, si_iu_h, ti_iu_h, s2_ui_h, att_ui_h, dv_ui_h,
                   s2_iu_h, att_iu_h, dv_iu_h, si_tab, as_tab, at_tab, si_f,
                   ti_b, s2_f, att_f, att8, bounce, dvc, dv_sh):
        c = lax.axis_index("c")
        s = lax.axis_index("s")
        iota16 = lax.broadcasted_iota(jnp.int32, (16,), 0)

        pltpu.sync_copy(z8_h.at[pl.ds(0, _WB)], bounce)
        pltpu.sync_copy(z8_h, att8)
        _for_each_wb_chunk(
            s, nwb, lambda r0: pltpu.sync_copy(
                bounce, dv_sh.at[pl.ds(r0, _WB)]))
        plsc.subcore_barrier()

        def process(as_h, at_h, si_h, ti_h, s2_h, att_h):
            pltpu.sync_copy(si_h.at[pl.ds(0, n)], si_tab)
            pltpu.sync_copy(as_h, as_tab)
            pltpu.sync_copy(at_h, at_tab)

            def super_body(q, _):
                sc = s + _NSUB * q

                @pl.when(sc < nsuper)
                def _():
                    off = sc * _SSA
                    pltpu.sync_copy(si_h.at[pl.ds(off, _SSA)], si_f)
                    pltpu.sync_copy(ti_h.at[pl.ds(off, _SSA)], s2_f)

                    @plsc.parallel_loop(0, _SSA // 16, unroll=2)
                    def _(i):
                        sl = pl.ds(16 * i, 16)
                        ti16 = s2_f[sl]
                        ti_b[i // 8, pl.ds(16 * (i % 8), 16)] = ti16
                        s2_16 = plsc.load_gather(si_tab, [si_f[sl]])
                        x = (plsc.load_gather(as_tab, [s2_16])
                             + plsc.load_gather(at_tab, [ti16]))
                        att16 = jnp.exp(jnp.where(x >= 0, x, 0.2 * x))
                        si_f[sl] = s2_16
                        att_f[sl] = att16
                    pltpu.sync_copy(si_f, s2_h.at[pl.ds(off, _SSA)])
                    pltpu.sync_copy(att_f, att_h.at[pl.ds(off, _SSA)])

                    def grp_body(g, _):
                        def cpy(i, _):
                            e16 = iota16 + 16 * i
                            att16 = att_f[pl.ds(_C * g + 16 * i, 16)]
                            plsc.store_scatter(
                                att8, [e16, jnp.zeros((16,), jnp.int32)],
                                att16)
                            return 0

                        lax.fori_loop(0, _C // 16, cpy, 0)
                        pltpu.sync_copy(att8, dv_sh.at[ti_b.at[g]], add=True)
                        return 0

                    lax.fori_loop(0, ngrp, grp_body, 0)
                return 0

            lax.fori_loop(0, qmax, super_body, 0)

        @pl.when(c == 0)
        def _():
            process(as_ui_h, at_ui_h, si_ui_h, ti_ui_h, s2_ui_h, att_ui_h)

        @pl.when(c == 1)
        def _():
            process(as_iu_h, at_iu_h, si_iu_h, ti_iu_h, s2_iu_h, att_iu_h)

        plsc.subcore_barrier()

        def compact_out(r0):
            pltpu.sync_copy(dv_sh.at[pl.ds(r0, _WB)], bounce)
            for j in range(_WB // 16):
                e16 = iota16 + 16 * j
                dvc[pl.ds(16 * j, 16)] = plsc.load_gather(
                    bounce, [e16, jnp.zeros((16,), jnp.int32)])

            @pl.when(c == 0)
            def _():
                pltpu.sync_copy(dvc, dv_ui_h.at[pl.ds(r0, _WB)])

            @pl.when(c == 1)
            def _():
                pltpu.sync_copy(dvc, dv_iu_h.at[pl.ds(r0, _WB)])

        _for_each_wb_chunk(s, nwb, compact_out)

    return att_kernel(zeros8, as_ui, at_ui, si_ui, ti_ui,
                      as_iu, at_iu, si_iu, ti_iu)


_CB = 128
_SUB = 10
_SS = _SUB * _CB


def _sc_agg_layer(sx_ui, att_ui, dv_ui, s2_ui, ti_ui,
                  sx_iu, att_iu, dv_iu, s2_iu, ti_iu):
    n = sx_ui.shape[0]
    e_cnt = s2_ui.shape[0]
    assert e_cnt % _SS == 0
    nsuper = e_cnt // _SS
    nwb = n // _WB
    qmax = (nsuper + _NSUB - 1) // _NSUB

    @functools.partial(
        pl.kernel, mesh=_MESH, compiler_params=_SC_PARAMS,
        out_type=[jax.ShapeDtypeStruct((n, 128), jnp.float32)] * 2,
        scratch_types=[
            pltpu.VMEM((n,), jnp.float32),
            pltpu.VMEM((_SUB, _CB), jnp.int32),
            pltpu.VMEM((_SS,), jnp.int32),
            pltpu.VMEM((_SS,), jnp.int32),
            pltpu.VMEM((_SS,), jnp.float32),
            pltpu.VMEM((_CB, 128), jnp.float32),
            pltpu.VMEM((_CB, 128), jnp.float32),
            pltpu.VMEM_SHARED((n, 128), jnp.float32),
            pltpu.SemaphoreType.DMA,
            pltpu.SemaphoreType.DMA,
        ])
    def agg_kernel(sx_ui_h, att_ui_h, dv_ui_h, s2_ui_h, ti_ui_h,
                   sx_iu_h, att_iu_h, dv_iu_h, s2_iu_h, ti_iu_h,
                   agg_ui_h, agg_iu_h, inv_tab, ti_b, ti_f, s2_f, w_b,
                   rows0, rows1, agg_sh, sem_g, sem_c):
        c = lax.axis_index("c")
        s = lax.axis_index("s")

        def zrow(r, _):
            for j in range(8):
                rows0[r, pl.ds(16 * j, 16)] = jnp.zeros((16,), jnp.float32)
            return 0

        lax.fori_loop(0, _WB, zrow, 0)
        _for_each_wb_chunk(
            s, nwb, lambda r0: pltpu.sync_copy(
                rows0.at[pl.ds(0, _WB)], agg_sh.at[pl.ds(r0, _WB)]))
        plsc.subcore_barrier()

        def process(sx_h, att_h, dv_h, s2_h, ti_h):
            pltpu.sync_copy(dv_h, inv_tab)

            def inv_body(j, _):
                sl = pl.ds(16 * j, 16)
                inv_tab[sl] = 1.0 / (inv_tab[sl] + 1e-06)
                return 0

            lax.fori_loop(0, n // 16, inv_body, 0)

            def on_buf(j, fn):
                @pl.when(j % 2 == 0)
                def _():
                    fn(rows0)

                @pl.when(j % 2 == 1)
                def _():
                    fn(rows1)

            def issue_gather(j):
                on_buf(j, lambda rb: pltpu.async_copy(
                    sx_h.at[s2_f.at[pl.ds(_CB * j, _CB)]], rb, sem_g))

            def wait_gather(j):
                on_buf(j, lambda rb: pltpu.make_async_copy(
                    sx_h.at[s2_f.at[pl.ds(_CB * j, _CB)]], rb, sem_g).wait())

            def issue_scatter(j):
                on_buf(j, lambda rb: pltpu.async_copy(
                    rb, agg_sh.at[ti_b.at[j]], sem_c, add=True))

            def wait_scatter(j):
                on_buf(j, lambda rb: pltpu.make_async_copy(
                    rb, agg_sh.at[ti_b.at[j]], sem_c).wait())

            def super_body(q, _):
                sc = s + _NSUB * q

                @pl.when(sc < nsuper)
                def _():
                    off = sc * _SS
                    pltpu.sync_copy(att_h.at[pl.ds(off, _SS)], w_b)
                    pltpu.sync_copy(s2_h.at[pl.ds(off, _SS)], s2_f)
                    pltpu.sync_copy(ti_h.at[pl.ds(off, _SS)], ti_f)

                    @plsc.parallel_loop(0, _SS // 16, unroll=2)
                    def _(i):
                        sl = pl.ds(16 * i, 16)
                        ti16 = ti_f[sl]
                        ti_b[i // 8, pl.ds(16 * (i % 8), 16)] = ti16
                        w_b[sl] = w_b[sl] * plsc.load_gather(
                            inv_tab, [ti16])

                    issue_gather(0)

                    def sub_body(j, _):
                        wait_gather(j)

                        @pl.when(j + 1 < _SUB)
                        def _():
                            @pl.when(j >= 1)
                            def _():
                                wait_scatter(j - 1)
                            issue_gather(j + 1)

                        def scale_in(rb):
                            @plsc.parallel_loop(0, _CB, unroll=4)
                            def _(e):
                                bvec = plsc.load_gather(
                                    w_b,
                                    [jnp.full((16,), _CB, jnp.int32) * j + e])
                                for r in range(8):
                                    sl = pl.ds(16 * r, 16)
                                    rb[e, sl] = rb[e, sl] * bvec

                        on_buf(j, scale_in)
                        issue_scatter(j)
                        return 0

                    lax.fori_loop(0, _SUB, sub_body, 0)
                    wait_scatter(_SUB - 2)
                    wait_scatter(_SUB - 1)
                return 0

            lax.fori_loop(0, qmax, super_body, 0)

        @pl.when(c == 0)
        def _():
            process(sx_ui_h, att_ui_h, dv_ui_h, s2_ui_h, ti_ui_h)

        @pl.when(c == 1)
        def _():
            process(sx_iu_h, att_iu_h, dv_iu_h, s2_iu_h, ti_iu_h)

        plsc.subcore_barrier()

        def wb(r0):
            pltpu.sync_copy(agg_sh.at[pl.ds(r0, _WB)], rows0.at[pl.ds(0, _WB)])

            @pl.when(c == 0)
            def _():
                pltpu.sync_copy(
                    rows0.at[pl.ds(0, _WB)], agg_ui_h.at[pl.ds(r0, _WB)])

            @pl.when(c == 1)
            def _():
                pltpu.sync_copy(
                    rows0.at[pl.ds(0, _WB)], agg_iu_h.at[pl.ds(r0, _WB)])

        _for_each_wb_chunk(s, nwb, wb)

    return agg_kernel(sx_ui, att_ui, dv_ui, s2_ui, ti_ui,
                      sx_iu, att_iu, dv_iu, s2_iu, ti_iu)


def _sc_conv_layer(zeros8, sx_ui, as_ui, at_ui, si_ui, ti_ui,
                   sx_iu, as_iu, at_iu, si_iu, ti_iu):
    s2_ui, att_ui, dv_ui, s2_iu, att_iu, dv_iu = _sc_att_layer(
        zeros8, as_ui, at_ui, si_ui, ti_ui, as_iu, at_iu, si_iu, ti_iu)
    return _sc_agg_layer(sx_ui, att_ui, dv_ui, s2_ui, ti_ui,
                         sx_iu, att_iu, dv_iu, s2_iu, ti_iu)


def kernel(x_user, x_item, edge_index_ui, edge_index_iu, params):
    n = x_user.shape[0]
    p = params
    si_ui, ti_ui = edge_index_ui[0], edge_index_ui[1]
    si_iu, ti_iu = edge_index_iu[0], edge_index_iu[1]

    w0 = [p["l0_ui_Wsrc"], p["l0_ui_Wdst"], p["l0_ui_a"],
          p["l0_iu_Wsrc"], p["l0_iu_Wdst"], p["l0_iu_a"]]
    (sx_ui, tx_ui, as_ui, at_ui, sx_iu, tx_iu, as_iu, at_iu) = (
        _run_layer_matmuls(_layer0_body, [x_user, x_item], w0, n))

    zeros8 = jnp.zeros((_C, 8), jnp.float32)
    agg_ui, agg_iu = _sc_conv_layer(
        zeros8, sx_ui, as_ui[:, 0], at_ui[:, 0], si_ui, ti_ui,
        sx_iu, as_iu[:, 0], at_iu[:, 0], si_iu, ti_iu)

    w1 = [p["l1_ui_Wsrc"], p["l1_ui_Wdst"], p["l1_ui_a"],
          p["l1_iu_Wsrc"], p["l1_iu_Wdst"], p["l1_iu_a"]]
    (sx_ui, tx_ui, as_ui, at_ui, sx_iu, tx_iu, as_iu, at_iu) = (
        _run_layer_matmuls(_layer1_body, [tx_ui, agg_ui, tx_iu, agg_iu],
                           w1, n))

    agg_ui, agg_iu = _sc_conv_layer(
        zeros8, sx_ui, as_ui[:, 0], at_ui[:, 0], si_ui, ti_ui,
        sx_iu, as_iu[:, 0], at_iu[:, 0], si_iu, ti_iu)

    wf = [p["lin_user_W"], p["lin_user_b"], p["lin_item_W"], p["lin_item_b"]]
    xu2, xi2, out_u, out_i = _run_final(
        [tx_ui, agg_ui, tx_iu, agg_iu], wf, n)
    return (xu2, xi2, out_u, out_i)
